# baseline (speedup 1.0000x reference)
import jax
import jax.numpy as jnp
from jax.experimental import pallas as pl

DIM = 192
HEADS = 8
CH = DIM // HEADS
HIDDEN = int(DIM * 2.66)
HW = 224
N = HW * HW
NT = 1024
GN = N // NT
EPS_LN = 1e-6
EPS_L2 = 1e-12

CT2 = 16
CT6 = 10


def _ln(x, w, b):
    u = jnp.mean(x, axis=0, keepdims=True)
    s = jnp.mean((x - u) * (x - u), axis=0, keepdims=True)
    return w * ((x - u) * jax.lax.rsqrt(s + EPS_LN)) + b


def _k1(x_ref, nw_ref, nb_ref, wqkv_ref, g1w_ref, g1b_ref, g2w_ref, g2b_ref,
        qkv0_ref, gsum_ref):
    xn = _ln(x_ref[...], nw_ref[...], nb_ref[...])
    qkv0_ref[...] = jnp.dot(wqkv_ref[...], xn, preferred_element_type=jnp.float32)
    gg = jnp.maximum(
        jnp.dot(g1w_ref[...], xn, preferred_element_type=jnp.float32) + g1b_ref[...], 0.0)
    gv = jax.nn.sigmoid(
        jnp.dot(g2w_ref[...], gg, preferred_element_type=jnp.float32) + g2b_ref[...])

    @pl.when(pl.program_id(0) == 0)
    def _():
        gsum_ref[...] = jnp.zeros_like(gsum_ref)

    gsum_ref[...] += jnp.sum(gv)


def _shift_h(x, s):
    c, h, w = x.shape
    z = jnp.zeros((c, 1, w), x.dtype)
    if s == 0:
        return x
    if s > 0:
        return jnp.concatenate([x[:, s:, :], z], axis=1)
    return jnp.concatenate([z, x[:, :s, :]], axis=1)


def _shift_w(x, s):
    c, h, w = x.shape
    z = jnp.zeros((c, h, 1), x.dtype)
    if s == 0:
        return x
    if s > 0:
        return jnp.concatenate([x[:, :, s:], z], axis=2)
    return jnp.concatenate([z, x[:, :, :s]], axis=2)


def _dw3x3(x, w):
    acc = None
    for dh in (-1, 0, 1):
        xh = _shift_h(x, dh)
        for dw in (-1, 0, 1):
            tap = (dh + 1) * 3 + (dw + 1)
            wv = w[:, tap:tap + 1][:, :, None]
            t = _shift_w(xh, dw) * wv
            acc = t if acc is None else acc + t
    return acc


def _k2(x_ref, w_ref, o_ref):
    o_ref[...] = _dw3x3(x_ref[...], w_ref[...])


def _k3(q_ref, k_ref, f_ref, st_ref):
    q = q_ref[...]
    k = k_ref[...]

    @pl.when(pl.program_id(0) == 0)
    def _():
        f_ref[...] = jnp.zeros_like(f_ref)
        st_ref[...] = jnp.zeros_like(st_ref)

    f_ref[...] += jax.lax.dot_general(
        q, k, (((1,), (1,)), ((), ())), preferred_element_type=jnp.float32)
    st_ref[:, 0:1] += jnp.sum(q * q, axis=1, keepdims=True)
    st_ref[:, 1:2] += jnp.sum(k * k, axis=1, keepdims=True)


def _k4(f_ref, st_ref, stt_ref, tv_ref, gs_ref, a_ref):
    dkf = jnp.clip(jnp.floor(CH * gs_ref[0, 0] / N), 1.0, float(CH))

    qn = jnp.maximum(jnp.sqrt(st_ref[:, 0:1]), EPS_L2)
    knt = jnp.maximum(jnp.sqrt(stt_ref[1:2, :]), EPS_L2)
    fn = f_ref[...] / qn / knt * tv_ref[...]

    rows = jax.lax.broadcasted_iota(jnp.int32, (DIM, CH), 0)
    a = jnp.zeros((DIM, CH), jnp.float32)
    for h in range(HEADS):
        in_h = (rows // CH) == h
        a = jnp.where(in_h, fn[:, h * CH:(h + 1) * CH], a)

    iota = jax.lax.broadcasted_iota(jnp.int32, (DIM, CH), 1)
    w = a
    keep = jnp.zeros((DIM, CH), jnp.bool_)
    neg = jnp.float32(-jnp.inf)
    for it in range(CH):
        m = jnp.max(w, axis=1, keepdims=True)
        eq = w == m
        midx = jnp.min(jnp.where(eq, iota, jnp.int32(CH)), axis=1, keepdims=True)
        first = iota == midx
        keep = keep | (first & (jnp.float32(it) < dkf))
        w = jnp.where(first, neg, w)

    s = jnp.where(keep, a, neg)
    mx = jnp.max(s, axis=1, keepdims=True)
    e = jnp.exp(s - mx)
    p = e / jnp.sum(e, axis=1, keepdims=True)

    a_ref[...] = jnp.zeros_like(a_ref)
    for h in range(HEADS):
        a_ref[h * CH:(h + 1) * CH, h * CH:(h + 1) * CH] = p[h * CH:(h + 1) * CH, :]


def _k5(x_ref, v_ref, abd_ref, pow_ref, nw_ref, nb_ref, piw_ref,
        x1_ref, h0_ref):
    av = jnp.dot(abd_ref[...], v_ref[...], preferred_element_type=jnp.float32)
    x1 = x_ref[...] + jnp.dot(pow_ref[...], av, preferred_element_type=jnp.float32)
    x1_ref[...] = x1
    xn1 = _ln(x1, nw_ref[...], nb_ref[...])
    h0_ref[...] = jnp.dot(piw_ref[...], xn1, preferred_element_type=jnp.float32)


def _k6(a_ref, b_ref, wa_ref, wb_ref, w1_ref, w2_ref, m_ref):
    xa = _dw3x3(a_ref[...], wa_ref[0])
    xb = _dw3x3(b_ref[...], wb_ref[0])
    x1 = jnp.tanh(_dw3x3(xa, w1_ref[0])) + xa
    x2 = jnp.tanh(_dw3x3(xb, w2_ref[0])) + xb
    m_ref[...] = x1 * x2


def _k7(x1_ref, m_ref, ipow_ref, o_ref):
    o_ref[...] = x1_ref[...] + jnp.dot(
        ipow_ref[...], m_ref[...], preferred_element_type=jnp.float32)


def kernel(x, norm_w, norm_b, temp, q_w, q_dw, kv_w, kv_dw, po_w, g1_w, g1_b,
           g2_w, g2_b, pi_w, dw_w, dw1_w, dw2_w, ipo_w):
    f32 = jnp.float32
    x2d = x.reshape(DIM, N)
    nw = norm_w.reshape(DIM, 1)
    nb = norm_b.reshape(DIM, 1)
    wqkv = jnp.concatenate([q_w.reshape(DIM, DIM), kv_w.reshape(2 * DIM, DIM)], axis=0)
    g1w = g1_w.reshape(DIM // 2, DIM)
    g1b = g1_b.reshape(DIM // 2, 1)
    g2w = g2_w.reshape(1, DIM // 2)
    g2b = g2_b.reshape(1, 1)
    wdw_qkv = jnp.concatenate([q_dw.reshape(DIM, 9), kv_dw.reshape(2 * DIM, 9)], axis=0)
    tvec = jnp.repeat(temp.reshape(HEADS), CH).reshape(DIM, 1)
    pow_ = po_w.reshape(DIM, DIM)
    piw = pi_w.reshape(2 * HIDDEN, DIM)
    dwa = dw_w.reshape(2 * HIDDEN, 9)[:HIDDEN].reshape(HIDDEN // CT6, CT6, 9)
    dwb = dw_w.reshape(2 * HIDDEN, 9)[HIDDEN:].reshape(HIDDEN // CT6, CT6, 9)
    dw1 = dw1_w.reshape(HIDDEN // CT6, CT6, 9)
    dw2 = dw2_w.reshape(HIDDEN // CT6, CT6, 9)
    ipow = ipo_w.reshape(DIM, HIDDEN)

    qkv0, gsum = pl.pallas_call(
        _k1,
        grid=(GN,),
        in_specs=[
            pl.BlockSpec((DIM, NT), lambda i: (0, i)),
            pl.BlockSpec((DIM, 1), lambda i: (0, 0)),
            pl.BlockSpec((DIM, 1), lambda i: (0, 0)),
            pl.BlockSpec((3 * DIM, DIM), lambda i: (0, 0)),
            pl.BlockSpec((DIM // 2, DIM), lambda i: (0, 0)),
            pl.BlockSpec((DIM // 2, 1), lambda i: (0, 0)),
            pl.BlockSpec((1, DIM // 2), lambda i: (0, 0)),
            pl.BlockSpec((1, 1), lambda i: (0, 0)),
        ],
        out_specs=[
            pl.BlockSpec((3 * DIM, NT), lambda i: (0, i)),
            pl.BlockSpec((8, 128), lambda i: (0, 0)),
        ],
        out_shape=[
            jax.ShapeDtypeStruct((3 * DIM, N), f32),
            jax.ShapeDtypeStruct((8, 128), f32),
        ],
    )(x2d, nw, nb, wqkv, g1w, g1b, g2w, g2b)

    qkv = pl.pallas_call(
        _k2,
        grid=(3 * DIM // CT2,),
        in_specs=[
            pl.BlockSpec((CT2, HW, HW), lambda c: (c, 0, 0)),
            pl.BlockSpec((CT2, 9), lambda c: (c, 0)),
        ],
        out_specs=pl.BlockSpec((CT2, HW, HW), lambda c: (c, 0, 0)),
        out_shape=jax.ShapeDtypeStruct((3 * DIM, HW, HW), f32),
    )(qkv0.reshape(3 * DIM, HW, HW), wdw_qkv)
    qkv2d = qkv.reshape(3 * DIM, N)

    gram, stats = pl.pallas_call(
        _k3,
        grid=(GN,),
        in_specs=[
            pl.BlockSpec((DIM, NT), lambda i: (0, i)),
            pl.BlockSpec((DIM, NT), lambda i: (1, i)),
        ],
        out_specs=[
            pl.BlockSpec((DIM, DIM), lambda i: (0, 0)),
            pl.BlockSpec((DIM, 128), lambda i: (0, 0)),
        ],
        out_shape=[
            jax.ShapeDtypeStruct((DIM, DIM), f32),
            jax.ShapeDtypeStruct((DIM, 128), f32),
        ],
    )(qkv2d, qkv2d)

    abd = pl.pallas_call(
        _k4,
        out_shape=jax.ShapeDtypeStruct((DIM, DIM), f32),
    )(gram, stats, stats.T, tvec, gsum)

    x1, h0 = pl.pallas_call(
        _k5,
        grid=(GN,),
        in_specs=[
            pl.BlockSpec((DIM, NT), lambda i: (0, i)),
            pl.BlockSpec((DIM, NT), lambda i: (2, i)),
            pl.BlockSpec((DIM, DIM), lambda i: (0, 0)),
            pl.BlockSpec((DIM, DIM), lambda i: (0, 0)),
            pl.BlockSpec((DIM, 1), lambda i: (0, 0)),
            pl.BlockSpec((DIM, 1), lambda i: (0, 0)),
            pl.BlockSpec((2 * HIDDEN, DIM), lambda i: (0, 0)),
        ],
        out_specs=[
            pl.BlockSpec((DIM, NT), lambda i: (0, i)),
            pl.BlockSpec((2 * HIDDEN, NT), lambda i: (0, i)),
        ],
        out_shape=[
            jax.ShapeDtypeStruct((DIM, N), f32),
            jax.ShapeDtypeStruct((2 * HIDDEN, N), f32),
        ],
    )(x2d, qkv2d, abd, pow_, nw, nb, piw)

    m = pl.pallas_call(
        _k6,
        grid=(HIDDEN // CT6,),
        in_specs=[
            pl.BlockSpec((CT6, HW, HW), lambda c: (c, 0, 0)),
            pl.BlockSpec((CT6, HW, HW), lambda c: (c + HIDDEN // CT6, 0, 0)),
            pl.BlockSpec((1, CT6, 9), lambda c: (c, 0, 0)),
            pl.BlockSpec((1, CT6, 9), lambda c: (c, 0, 0)),
            pl.BlockSpec((1, CT6, 9), lambda c: (c, 0, 0)),
            pl.BlockSpec((1, CT6, 9), lambda c: (c, 0, 0)),
        ],
        out_specs=pl.BlockSpec((CT6, HW, HW), lambda c: (c, 0, 0)),
        out_shape=jax.ShapeDtypeStruct((HIDDEN, HW, HW), f32),
    )(h0.reshape(2 * HIDDEN, HW, HW), h0.reshape(2 * HIDDEN, HW, HW),
      dwa, dwb, dw1, dw2)

    out = pl.pallas_call(
        _k7,
        grid=(GN,),
        in_specs=[
            pl.BlockSpec((DIM, NT), lambda i: (0, i)),
            pl.BlockSpec((HIDDEN, NT), lambda i: (0, i)),
            pl.BlockSpec((DIM, HIDDEN), lambda i: (0, 0)),
        ],
        out_specs=pl.BlockSpec((DIM, NT), lambda i: (0, i)),
        out_shape=jax.ShapeDtypeStruct((DIM, N), f32),
    )(x1, m.reshape(HIDDEN, N), ipow)

    return out.reshape(1, DIM, HW, HW)


# speedup vs baseline: 1.1915x; 1.1915x over previous
import jax
import jax.numpy as jnp
from jax.experimental import pallas as pl
from jax.experimental.pallas import tpu as pltpu

DIM = 192
HEADS = 8
CH = DIM // HEADS
HIDDEN = int(DIM * 2.66)
HW = 224
N = HW * HW
NT = 1024
GN = N // NT
EPS_LN = 1e-6
EPS_L2 = 1e-12

CT2 = 16
CT6 = 10

_PAR = pltpu.CompilerParams(dimension_semantics=("parallel",))
_ARB = pltpu.CompilerParams(dimension_semantics=("arbitrary",))


def _ln(x, w, b):
    u = jnp.mean(x, axis=0, keepdims=True)
    s = jnp.mean((x - u) * (x - u), axis=0, keepdims=True)
    return w * ((x - u) * jax.lax.rsqrt(s + EPS_LN)) + b


def _k1(x_ref, nw_ref, nb_ref, wqkv_ref, g1w_ref, g1b_ref, g2w_ref, g2b_ref,
        qkv0_ref, gsum_ref):
    xn = _ln(x_ref[...], nw_ref[...], nb_ref[...])
    qkv0_ref[...] = jnp.dot(wqkv_ref[...], xn, preferred_element_type=jnp.float32)
    gg = jnp.maximum(
        jnp.dot(g1w_ref[...], xn, preferred_element_type=jnp.float32) + g1b_ref[...], 0.0)
    gv = jax.nn.sigmoid(
        jnp.dot(g2w_ref[...], gg, preferred_element_type=jnp.float32) + g2b_ref[...])

    @pl.when(pl.program_id(0) == 0)
    def _():
        gsum_ref[...] = jnp.zeros_like(gsum_ref)

    gsum_ref[...] += jnp.sum(gv)


STRIP = 8


def _dw_strips(in_ref, w):
    c, hh, ww = in_ref.shape
    wv = [w[:, t:t + 1][:, :, None] for t in range(9)]
    for s in range(hh // STRIP):
        r0 = s * STRIP
        lo = max(r0 - 1, 0)
        hi = min(r0 + STRIP + 1, hh)
        xs = in_ref[:, lo:hi, :]
        zr = jnp.zeros((c, 1, ww), xs.dtype)
        if r0 == 0:
            xs = jnp.concatenate([zr, xs], axis=1)
        if hi == hh:
            xs = jnp.concatenate([xs, zr], axis=1)
        win = [xs[:, dh:dh + STRIP, :] for dh in range(3)]
        ys = []
        for dw in range(3):
            y = win[0] * wv[dw]
            y = y + win[1] * wv[3 + dw]
            y = y + win[2] * wv[6 + dw]
            ys.append(y)
        zc = jnp.zeros((c, STRIP, 1), xs.dtype)
        left = jnp.concatenate([ys[2][:, :, 1:], zc], axis=2)
        right = jnp.concatenate([zc, ys[0][:, :, :-1]], axis=2)
        yield r0, ys[1] + left + right


def _k2(x_ref, w_ref, o_ref):
    w = w_ref[...]
    for r0, acc in _dw_strips(x_ref, w):
        o_ref[:, r0:r0 + STRIP, :] = acc


def _k3(q_ref, k_ref, f_ref, st_ref):
    q = q_ref[...]
    k = k_ref[...]

    @pl.when(pl.program_id(0) == 0)
    def _():
        f_ref[...] = jnp.zeros_like(f_ref)
        st_ref[...] = jnp.zeros_like(st_ref)

    f_ref[...] += jax.lax.dot_general(
        q, k, (((1,), (1,)), ((), ())), preferred_element_type=jnp.float32)
    st_ref[:, 0:1] += jnp.sum(q * q, axis=1, keepdims=True)
    st_ref[:, 1:2] += jnp.sum(k * k, axis=1, keepdims=True)


def _k4(f_ref, st_ref, stt_ref, tv_ref, gs_ref, a_ref):
    dkf = jnp.clip(jnp.floor(CH * gs_ref[0, 0] / N), 1.0, float(CH))

    qn = jnp.maximum(jnp.sqrt(st_ref[:, 0:1]), EPS_L2)
    knt = jnp.maximum(jnp.sqrt(stt_ref[1:2, :]), EPS_L2)
    fn = f_ref[...] / qn / knt * tv_ref[...]

    rows = jax.lax.broadcasted_iota(jnp.int32, (DIM, CH), 0)
    a = jnp.zeros((DIM, CH), jnp.float32)
    for h in range(HEADS):
        in_h = (rows // CH) == h
        a = jnp.where(in_h, fn[:, h * CH:(h + 1) * CH], a)

    iota = jax.lax.broadcasted_iota(jnp.int32, (DIM, CH), 1)
    w = a
    keep = jnp.zeros((DIM, CH), jnp.bool_)
    neg = jnp.float32(-jnp.inf)
    for it in range(CH):
        m = jnp.max(w, axis=1, keepdims=True)
        eq = w == m
        midx = jnp.min(jnp.where(eq, iota, jnp.int32(CH)), axis=1, keepdims=True)
        first = iota == midx
        keep = keep | (first & (jnp.float32(it) < dkf))
        w = jnp.where(first, neg, w)

    s = jnp.where(keep, a, neg)
    mx = jnp.max(s, axis=1, keepdims=True)
    e = jnp.exp(s - mx)
    p = e / jnp.sum(e, axis=1, keepdims=True)

    a_ref[...] = jnp.zeros_like(a_ref)
    for h in range(HEADS):
        a_ref[h * CH:(h + 1) * CH, h * CH:(h + 1) * CH] = p[h * CH:(h + 1) * CH, :]


def _k5(x_ref, v_ref, abd_ref, pow_ref, nw_ref, nb_ref, piw_ref,
        x1_ref, h0_ref):
    av = jnp.dot(abd_ref[...], v_ref[...], preferred_element_type=jnp.float32)
    x1 = x_ref[...] + jnp.dot(pow_ref[...], av, preferred_element_type=jnp.float32)
    x1_ref[...] = x1
    xn1 = _ln(x1, nw_ref[...], nb_ref[...])
    h0_ref[...] = jnp.dot(piw_ref[...], xn1, preferred_element_type=jnp.float32)


def _k6(a_ref, b_ref, wa_ref, wb_ref, w1_ref, w2_ref, m_ref, sa_ref, sb_ref):
    wa = wa_ref[0]
    wb = wb_ref[0]
    for r0, acc in _dw_strips(a_ref, wa):
        sa_ref[:, r0:r0 + STRIP, :] = acc
    for r0, acc in _dw_strips(b_ref, wb):
        sb_ref[:, r0:r0 + STRIP, :] = acc
    w1 = w1_ref[0]
    w2 = w2_ref[0]
    for (r0, t1), (_, t2) in zip(_dw_strips(sa_ref, w1), _dw_strips(sb_ref, w2)):
        v1 = jnp.tanh(t1) + sa_ref[:, r0:r0 + STRIP, :]
        v2 = jnp.tanh(t2) + sb_ref[:, r0:r0 + STRIP, :]
        m_ref[:, r0:r0 + STRIP, :] = v1 * v2


def _k7(x1_ref, m_ref, ipow_ref, o_ref):
    o_ref[...] = x1_ref[...] + jnp.dot(
        ipow_ref[...], m_ref[...], preferred_element_type=jnp.float32)


def kernel(x, norm_w, norm_b, temp, q_w, q_dw, kv_w, kv_dw, po_w, g1_w, g1_b,
           g2_w, g2_b, pi_w, dw_w, dw1_w, dw2_w, ipo_w):
    f32 = jnp.float32
    x2d = x.reshape(DIM, N)
    nw = norm_w.reshape(DIM, 1)
    nb = norm_b.reshape(DIM, 1)
    wqkv = jnp.concatenate([q_w.reshape(DIM, DIM), kv_w.reshape(2 * DIM, DIM)], axis=0)
    g1w = g1_w.reshape(DIM // 2, DIM)
    g1b = g1_b.reshape(DIM // 2, 1)
    g2w = g2_w.reshape(1, DIM // 2)
    g2b = g2_b.reshape(1, 1)
    wdw_qkv = jnp.concatenate([q_dw.reshape(DIM, 9), kv_dw.reshape(2 * DIM, 9)], axis=0)
    tvec = jnp.repeat(temp.reshape(HEADS), CH).reshape(DIM, 1)
    pow_ = po_w.reshape(DIM, DIM)
    piw = pi_w.reshape(2 * HIDDEN, DIM)
    dwa = dw_w.reshape(2 * HIDDEN, 9)[:HIDDEN].reshape(HIDDEN // CT6, CT6, 9)
    dwb = dw_w.reshape(2 * HIDDEN, 9)[HIDDEN:].reshape(HIDDEN // CT6, CT6, 9)
    dw1 = dw1_w.reshape(HIDDEN // CT6, CT6, 9)
    dw2 = dw2_w.reshape(HIDDEN // CT6, CT6, 9)
    ipow = ipo_w.reshape(DIM, HIDDEN)

    qkv0, gsum = pl.pallas_call(
        _k1,
        grid=(GN,),
        in_specs=[
            pl.BlockSpec((DIM, NT), lambda i: (0, i)),
            pl.BlockSpec((DIM, 1), lambda i: (0, 0)),
            pl.BlockSpec((DIM, 1), lambda i: (0, 0)),
            pl.BlockSpec((3 * DIM, DIM), lambda i: (0, 0)),
            pl.BlockSpec((DIM // 2, DIM), lambda i: (0, 0)),
            pl.BlockSpec((DIM // 2, 1), lambda i: (0, 0)),
            pl.BlockSpec((1, DIM // 2), lambda i: (0, 0)),
            pl.BlockSpec((1, 1), lambda i: (0, 0)),
        ],
        out_specs=[
            pl.BlockSpec((3 * DIM, NT), lambda i: (0, i)),
            pl.BlockSpec((8, 128), lambda i: (0, 0)),
        ],
        out_shape=[
            jax.ShapeDtypeStruct((3 * DIM, N), f32),
            jax.ShapeDtypeStruct((8, 128), f32),
        ],
        compiler_params=_ARB,
    )(x2d, nw, nb, wqkv, g1w, g1b, g2w, g2b)

    qkv = pl.pallas_call(
        _k2,
        grid=(3 * DIM // CT2,),
        in_specs=[
            pl.BlockSpec((CT2, HW, HW), lambda c: (c, 0, 0)),
            pl.BlockSpec((CT2, 9), lambda c: (c, 0)),
        ],
        out_specs=pl.BlockSpec((CT2, HW, HW), lambda c: (c, 0, 0)),
        out_shape=jax.ShapeDtypeStruct((3 * DIM, HW, HW), f32),
        compiler_params=_PAR,
    )(qkv0.reshape(3 * DIM, HW, HW), wdw_qkv)
    qkv2d = qkv.reshape(3 * DIM, N)

    gram, stats = pl.pallas_call(
        _k3,
        grid=(GN,),
        in_specs=[
            pl.BlockSpec((DIM, NT), lambda i: (0, i)),
            pl.BlockSpec((DIM, NT), lambda i: (1, i)),
        ],
        out_specs=[
            pl.BlockSpec((DIM, DIM), lambda i: (0, 0)),
            pl.BlockSpec((DIM, 128), lambda i: (0, 0)),
        ],
        out_shape=[
            jax.ShapeDtypeStruct((DIM, DIM), f32),
            jax.ShapeDtypeStruct((DIM, 128), f32),
        ],
        compiler_params=_ARB,
    )(qkv2d, qkv2d)

    abd = pl.pallas_call(
        _k4,
        out_shape=jax.ShapeDtypeStruct((DIM, DIM), f32),
    )(gram, stats, stats.T, tvec, gsum)

    x1, h0 = pl.pallas_call(
        _k5,
        grid=(GN,),
        in_specs=[
            pl.BlockSpec((DIM, NT), lambda i: (0, i)),
            pl.BlockSpec((DIM, NT), lambda i: (2, i)),
            pl.BlockSpec((DIM, DIM), lambda i: (0, 0)),
            pl.BlockSpec((DIM, DIM), lambda i: (0, 0)),
            pl.BlockSpec((DIM, 1), lambda i: (0, 0)),
            pl.BlockSpec((DIM, 1), lambda i: (0, 0)),
            pl.BlockSpec((2 * HIDDEN, DIM), lambda i: (0, 0)),
        ],
        out_specs=[
            pl.BlockSpec((DIM, NT), lambda i: (0, i)),
            pl.BlockSpec((2 * HIDDEN, NT), lambda i: (0, i)),
        ],
        out_shape=[
            jax.ShapeDtypeStruct((DIM, N), f32),
            jax.ShapeDtypeStruct((2 * HIDDEN, N), f32),
        ],
        compiler_params=_PAR,
    )(x2d, qkv2d, abd, pow_, nw, nb, piw)

    m = pl.pallas_call(
        _k6,
        grid=(HIDDEN // CT6,),
        in_specs=[
            pl.BlockSpec((CT6, HW, HW), lambda c: (c, 0, 0)),
            pl.BlockSpec((CT6, HW, HW), lambda c: (c + HIDDEN // CT6, 0, 0)),
            pl.BlockSpec((1, CT6, 9), lambda c: (c, 0, 0)),
            pl.BlockSpec((1, CT6, 9), lambda c: (c, 0, 0)),
            pl.BlockSpec((1, CT6, 9), lambda c: (c, 0, 0)),
            pl.BlockSpec((1, CT6, 9), lambda c: (c, 0, 0)),
        ],
        out_specs=pl.BlockSpec((CT6, HW, HW), lambda c: (c, 0, 0)),
        out_shape=jax.ShapeDtypeStruct((HIDDEN, HW, HW), f32),
        scratch_shapes=[
            pltpu.VMEM((CT6, HW, HW), f32),
            pltpu.VMEM((CT6, HW, HW), f32),
        ],
        compiler_params=_PAR,
    )(h0.reshape(2 * HIDDEN, HW, HW), h0.reshape(2 * HIDDEN, HW, HW),
      dwa, dwb, dw1, dw2)

    out = pl.pallas_call(
        _k7,
        grid=(GN,),
        in_specs=[
            pl.BlockSpec((DIM, NT), lambda i: (0, i)),
            pl.BlockSpec((HIDDEN, NT), lambda i: (0, i)),
            pl.BlockSpec((DIM, HIDDEN), lambda i: (0, 0)),
        ],
        out_specs=pl.BlockSpec((DIM, NT), lambda i: (0, i)),
        out_shape=jax.ShapeDtypeStruct((DIM, N), f32),
        compiler_params=_PAR,
    )(x1, m.reshape(HIDDEN, N), ipow)

    return out.reshape(1, DIM, HW, HW)


# speedup vs baseline: 1.2091x; 1.0148x over previous
import jax
import jax.numpy as jnp
from jax.experimental import pallas as pl
from jax.experimental.pallas import tpu as pltpu

DIM = 192
HEADS = 8
CH = DIM // HEADS
HIDDEN = int(DIM * 2.66)
HW = 224
N = HW * HW
NT = 1024
GN = N // NT
EPS_LN = 1e-6
EPS_L2 = 1e-12

CT2 = 16
CT6 = 10

_PAR = pltpu.CompilerParams(dimension_semantics=("parallel",))
_ARB = pltpu.CompilerParams(dimension_semantics=("arbitrary",))


def _ln(x, w, b):
    u = jnp.mean(x, axis=0, keepdims=True)
    s = jnp.mean((x - u) * (x - u), axis=0, keepdims=True)
    return w * ((x - u) * jax.lax.rsqrt(s + EPS_LN)) + b


def _k1(x_ref, nw_ref, nb_ref, wqk_ref, wv_ref, g1w_ref, g1b_ref, g2w_ref, g2b_ref,
        qk0_ref, v0_ref, gsum_ref):
    xn = _ln(x_ref[...], nw_ref[...], nb_ref[...])
    qk0_ref[...] = jnp.dot(wqk_ref[...], xn, preferred_element_type=jnp.float32)
    v0_ref[...] = jnp.dot(wv_ref[...], xn,
                          preferred_element_type=jnp.float32).astype(jnp.bfloat16)
    gg = jnp.maximum(
        jnp.dot(g1w_ref[...], xn, preferred_element_type=jnp.float32) + g1b_ref[...], 0.0)
    gv = jax.nn.sigmoid(
        jnp.dot(g2w_ref[...], gg, preferred_element_type=jnp.float32) + g2b_ref[...])

    @pl.when(pl.program_id(0) == 0)
    def _():
        gsum_ref[...] = jnp.zeros_like(gsum_ref)

    gsum_ref[...] += jnp.sum(gv)


STRIP = 8


def _dw_strips(in_ref, w):
    c, hh, ww = in_ref.shape
    wv = [w[:, t:t + 1][:, :, None] for t in range(9)]
    for s in range(hh // STRIP):
        r0 = s * STRIP
        lo = max(r0 - 1, 0)
        hi = min(r0 + STRIP + 1, hh)
        xs = in_ref[:, lo:hi, :].astype(jnp.float32)
        zr = jnp.zeros((c, 1, ww), xs.dtype)
        if r0 == 0:
            xs = jnp.concatenate([zr, xs], axis=1)
        if hi == hh:
            xs = jnp.concatenate([xs, zr], axis=1)
        win = [xs[:, dh:dh + STRIP, :] for dh in range(3)]
        ys = []
        for dw in range(3):
            y = win[0] * wv[dw]
            y = y + win[1] * wv[3 + dw]
            y = y + win[2] * wv[6 + dw]
            ys.append(y)
        zc = jnp.zeros((c, STRIP, 1), xs.dtype)
        left = jnp.concatenate([ys[2][:, :, 1:], zc], axis=2)
        right = jnp.concatenate([zc, ys[0][:, :, :-1]], axis=2)
        yield r0, ys[1] + left + right


def _k2(x_ref, w_ref, o_ref):
    w = w_ref[...]
    for r0, acc in _dw_strips(x_ref, w):
        o_ref[:, r0:r0 + STRIP, :] = acc.astype(o_ref.dtype)


def _k3(q_ref, k_ref, f_ref, st_ref):
    q = q_ref[...]
    k = k_ref[...]

    @pl.when(pl.program_id(0) == 0)
    def _():
        f_ref[...] = jnp.zeros_like(f_ref)
        st_ref[...] = jnp.zeros_like(st_ref)

    f_ref[...] += jax.lax.dot_general(
        q, k, (((1,), (1,)), ((), ())), preferred_element_type=jnp.float32)
    st_ref[:, 0:1] += jnp.sum(q * q, axis=1, keepdims=True)
    st_ref[:, 1:2] += jnp.sum(k * k, axis=1, keepdims=True)


def _k4(f_ref, st_ref, stt_ref, tv_ref, gs_ref, a_ref):
    dkf = jnp.clip(jnp.floor(CH * gs_ref[0, 0] / N), 1.0, float(CH))

    qn = jnp.maximum(jnp.sqrt(st_ref[:, 0:1]), EPS_L2)
    knt = jnp.maximum(jnp.sqrt(stt_ref[1:2, :]), EPS_L2)
    fn = f_ref[...] / qn / knt * tv_ref[...]

    rows = jax.lax.broadcasted_iota(jnp.int32, (DIM, CH), 0)
    a = jnp.zeros((DIM, CH), jnp.float32)
    for h in range(HEADS):
        in_h = (rows // CH) == h
        a = jnp.where(in_h, fn[:, h * CH:(h + 1) * CH], a)

    iota = jax.lax.broadcasted_iota(jnp.int32, (DIM, CH), 1)
    w = a
    keep = jnp.zeros((DIM, CH), jnp.bool_)
    neg = jnp.float32(-jnp.inf)
    for it in range(CH):
        m = jnp.max(w, axis=1, keepdims=True)
        eq = w == m
        midx = jnp.min(jnp.where(eq, iota, jnp.int32(CH)), axis=1, keepdims=True)
        first = iota == midx
        keep = keep | (first & (jnp.float32(it) < dkf))
        w = jnp.where(first, neg, w)

    s = jnp.where(keep, a, neg)
    mx = jnp.max(s, axis=1, keepdims=True)
    e = jnp.exp(s - mx)
    p = e / jnp.sum(e, axis=1, keepdims=True)

    a_ref[...] = jnp.zeros_like(a_ref)
    for h in range(HEADS):
        a_ref[h * CH:(h + 1) * CH, h * CH:(h + 1) * CH] = p[h * CH:(h + 1) * CH, :]


def _k5(x_ref, v_ref, abd_ref, pow_ref, nw_ref, nb_ref, piw_ref,
        x1_ref, h0_ref):
    av = jnp.dot(abd_ref[...].astype(jnp.bfloat16), v_ref[...],
                 preferred_element_type=jnp.float32)
    x1 = x_ref[...] + jnp.dot(pow_ref[...], av, preferred_element_type=jnp.float32)
    x1_ref[...] = x1
    xn1 = _ln(x1, nw_ref[...], nb_ref[...])
    h0_ref[...] = jnp.dot(piw_ref[...], xn1.astype(jnp.bfloat16),
                          preferred_element_type=jnp.float32).astype(jnp.bfloat16)


def _k6(a_ref, b_ref, wa_ref, wb_ref, w1_ref, w2_ref, m_ref, sa_ref, sb_ref):
    wa = wa_ref[0]
    wb = wb_ref[0]
    for r0, acc in _dw_strips(a_ref, wa):
        sa_ref[:, r0:r0 + STRIP, :] = acc
    for r0, acc in _dw_strips(b_ref, wb):
        sb_ref[:, r0:r0 + STRIP, :] = acc
    w1 = w1_ref[0]
    w2 = w2_ref[0]
    for (r0, t1), (_, t2) in zip(_dw_strips(sa_ref, w1), _dw_strips(sb_ref, w2)):
        v1 = jnp.tanh(t1) + sa_ref[:, r0:r0 + STRIP, :]
        v2 = jnp.tanh(t2) + sb_ref[:, r0:r0 + STRIP, :]
        m_ref[:, r0:r0 + STRIP, :] = (v1 * v2).astype(m_ref.dtype)


def _k7(x1_ref, m_ref, ipow_ref, o_ref):
    o_ref[...] = x1_ref[...] + jnp.dot(
        ipow_ref[...], m_ref[...], preferred_element_type=jnp.float32)


def kernel(x, norm_w, norm_b, temp, q_w, q_dw, kv_w, kv_dw, po_w, g1_w, g1_b,
           g2_w, g2_b, pi_w, dw_w, dw1_w, dw2_w, ipo_w):
    f32 = jnp.float32
    x2d = x.reshape(DIM, N)
    nw = norm_w.reshape(DIM, 1)
    nb = norm_b.reshape(DIM, 1)
    kvw = kv_w.reshape(2 * DIM, DIM)
    wqk = jnp.concatenate([q_w.reshape(DIM, DIM), kvw[:DIM]], axis=0)
    wv = kvw[DIM:]
    g1w = g1_w.reshape(DIM // 2, DIM)
    g1b = g1_b.reshape(DIM // 2, 1)
    g2w = g2_w.reshape(1, DIM // 2)
    g2b = g2_b.reshape(1, 1)
    kvdw = kv_dw.reshape(2 * DIM, 9)
    wdw_qk = jnp.concatenate([q_dw.reshape(DIM, 9), kvdw[:DIM]], axis=0)
    wdw_v = kvdw[DIM:]
    tvec = jnp.repeat(temp.reshape(HEADS), CH).reshape(DIM, 1)
    pow_ = po_w.reshape(DIM, DIM)
    piw = pi_w.reshape(2 * HIDDEN, DIM).astype(jnp.bfloat16)
    dwa = dw_w.reshape(2 * HIDDEN, 9)[:HIDDEN].reshape(HIDDEN // CT6, CT6, 9)
    dwb = dw_w.reshape(2 * HIDDEN, 9)[HIDDEN:].reshape(HIDDEN // CT6, CT6, 9)
    dw1 = dw1_w.reshape(HIDDEN // CT6, CT6, 9)
    dw2 = dw2_w.reshape(HIDDEN // CT6, CT6, 9)
    ipow = ipo_w.reshape(DIM, HIDDEN).astype(jnp.bfloat16)

    qk0, v0, gsum = pl.pallas_call(
        _k1,
        grid=(GN,),
        in_specs=[
            pl.BlockSpec((DIM, NT), lambda i: (0, i)),
            pl.BlockSpec((DIM, 1), lambda i: (0, 0)),
            pl.BlockSpec((DIM, 1), lambda i: (0, 0)),
            pl.BlockSpec((2 * DIM, DIM), lambda i: (0, 0)),
            pl.BlockSpec((DIM, DIM), lambda i: (0, 0)),
            pl.BlockSpec((DIM // 2, DIM), lambda i: (0, 0)),
            pl.BlockSpec((DIM // 2, 1), lambda i: (0, 0)),
            pl.BlockSpec((1, DIM // 2), lambda i: (0, 0)),
            pl.BlockSpec((1, 1), lambda i: (0, 0)),
        ],
        out_specs=[
            pl.BlockSpec((2 * DIM, NT), lambda i: (0, i)),
            pl.BlockSpec((DIM, NT), lambda i: (0, i)),
            pl.BlockSpec((8, 128), lambda i: (0, 0)),
        ],
        out_shape=[
            jax.ShapeDtypeStruct((2 * DIM, N), f32),
            jax.ShapeDtypeStruct((DIM, N), jnp.bfloat16),
            jax.ShapeDtypeStruct((8, 128), f32),
        ],
        compiler_params=_ARB,
    )(x2d, nw, nb, wqk, wv, g1w, g1b, g2w, g2b)

    qk = pl.pallas_call(
        _k2,
        grid=(2 * DIM // CT2,),
        in_specs=[
            pl.BlockSpec((CT2, HW, HW), lambda c: (c, 0, 0)),
            pl.BlockSpec((CT2, 9), lambda c: (c, 0)),
        ],
        out_specs=pl.BlockSpec((CT2, HW, HW), lambda c: (c, 0, 0)),
        out_shape=jax.ShapeDtypeStruct((2 * DIM, HW, HW), f32),
        compiler_params=_PAR,
    )(qk0.reshape(2 * DIM, HW, HW), wdw_qk)
    qk2d = qk.reshape(2 * DIM, N)

    v = pl.pallas_call(
        _k2,
        grid=(DIM // CT2,),
        in_specs=[
            pl.BlockSpec((CT2, HW, HW), lambda c: (c, 0, 0)),
            pl.BlockSpec((CT2, 9), lambda c: (c, 0)),
        ],
        out_specs=pl.BlockSpec((CT2, HW, HW), lambda c: (c, 0, 0)),
        out_shape=jax.ShapeDtypeStruct((DIM, HW, HW), jnp.bfloat16),
        compiler_params=_PAR,
    )(v0.reshape(DIM, HW, HW), wdw_v)
    v2d = v.reshape(DIM, N)

    gram, stats = pl.pallas_call(
        _k3,
        grid=(GN,),
        in_specs=[
            pl.BlockSpec((DIM, NT), lambda i: (0, i)),
            pl.BlockSpec((DIM, NT), lambda i: (1, i)),
        ],
        out_specs=[
            pl.BlockSpec((DIM, DIM), lambda i: (0, 0)),
            pl.BlockSpec((DIM, 128), lambda i: (0, 0)),
        ],
        out_shape=[
            jax.ShapeDtypeStruct((DIM, DIM), f32),
            jax.ShapeDtypeStruct((DIM, 128), f32),
        ],
        compiler_params=_ARB,
    )(qk2d, qk2d)

    abd = pl.pallas_call(
        _k4,
        out_shape=jax.ShapeDtypeStruct((DIM, DIM), f32),
    )(gram, stats, stats.T, tvec, gsum)

    x1, h0 = pl.pallas_call(
        _k5,
        grid=(GN,),
        in_specs=[
            pl.BlockSpec((DIM, NT), lambda i: (0, i)),
            pl.BlockSpec((DIM, NT), lambda i: (0, i)),
            pl.BlockSpec((DIM, DIM), lambda i: (0, 0)),
            pl.BlockSpec((DIM, DIM), lambda i: (0, 0)),
            pl.BlockSpec((DIM, 1), lambda i: (0, 0)),
            pl.BlockSpec((DIM, 1), lambda i: (0, 0)),
            pl.BlockSpec((2 * HIDDEN, DIM), lambda i: (0, 0)),
        ],
        out_specs=[
            pl.BlockSpec((DIM, NT), lambda i: (0, i)),
            pl.BlockSpec((2 * HIDDEN, NT), lambda i: (0, i)),
        ],
        out_shape=[
            jax.ShapeDtypeStruct((DIM, N), f32),
            jax.ShapeDtypeStruct((2 * HIDDEN, N), jnp.bfloat16),
        ],
        compiler_params=_PAR,
    )(x2d, v2d, abd, pow_, nw, nb, piw)

    m = pl.pallas_call(
        _k6,
        grid=(HIDDEN // CT6,),
        in_specs=[
            pl.BlockSpec((CT6, HW, HW), lambda c: (c, 0, 0)),
            pl.BlockSpec((CT6, HW, HW), lambda c: (c + HIDDEN // CT6, 0, 0)),
            pl.BlockSpec((1, CT6, 9), lambda c: (c, 0, 0)),
            pl.BlockSpec((1, CT6, 9), lambda c: (c, 0, 0)),
            pl.BlockSpec((1, CT6, 9), lambda c: (c, 0, 0)),
            pl.BlockSpec((1, CT6, 9), lambda c: (c, 0, 0)),
        ],
        out_specs=pl.BlockSpec((CT6, HW, HW), lambda c: (c, 0, 0)),
        out_shape=jax.ShapeDtypeStruct((HIDDEN, HW, HW), jnp.bfloat16),
        scratch_shapes=[
            pltpu.VMEM((CT6, HW, HW), f32),
            pltpu.VMEM((CT6, HW, HW), f32),
        ],
        compiler_params=_PAR,
    )(h0.reshape(2 * HIDDEN, HW, HW), h0.reshape(2 * HIDDEN, HW, HW),
      dwa, dwb, dw1, dw2)

    out = pl.pallas_call(
        _k7,
        grid=(GN,),
        in_specs=[
            pl.BlockSpec((DIM, NT), lambda i: (0, i)),
            pl.BlockSpec((HIDDEN, NT), lambda i: (0, i)),
            pl.BlockSpec((DIM, HIDDEN), lambda i: (0, 0)),
        ],
        out_specs=pl.BlockSpec((DIM, NT), lambda i: (0, i)),
        out_shape=jax.ShapeDtypeStruct((DIM, N), f32),
        compiler_params=_PAR,
    )(x1, m.reshape(HIDDEN, N), ipow)

    return out.reshape(1, DIM, HW, HW)


# speedup vs baseline: 1.5331x; 1.2679x over previous
import jax
import jax.numpy as jnp
from jax.experimental import pallas as pl
from jax.experimental.pallas import tpu as pltpu

DIM = 192
HEADS = 8
CH = DIM // HEADS
HIDDEN = int(DIM * 2.66)
HW = 224
N = HW * HW
NT = 1024
GN = N // NT
EPS_LN = 1e-6
EPS_L2 = 1e-12

CT2 = 16
CT6 = 10

_PAR = pltpu.CompilerParams(dimension_semantics=("parallel",))
_ARB = pltpu.CompilerParams(dimension_semantics=("arbitrary",))


def _ln(x, w, b):
    u = jnp.mean(x, axis=0, keepdims=True)
    s = jnp.mean((x - u) * (x - u), axis=0, keepdims=True)
    return w * ((x - u) * jax.lax.rsqrt(s + EPS_LN)) + b


def _k1(x_ref, nw_ref, nb_ref, wqk_ref, wv_ref, g1w_ref, g1b_ref, g2w_ref, g2b_ref,
        qk0_ref, v0_ref, gsum_ref):
    xn = _ln(x_ref[...], nw_ref[...], nb_ref[...])
    qk0_ref[...] = jnp.dot(wqk_ref[...], xn, preferred_element_type=jnp.float32)
    v0_ref[...] = jnp.dot(wv_ref[...], xn,
                          preferred_element_type=jnp.float32).astype(jnp.bfloat16)
    gg = jnp.maximum(
        jnp.dot(g1w_ref[...], xn, preferred_element_type=jnp.float32) + g1b_ref[...], 0.0)
    gv = jax.nn.sigmoid(
        jnp.dot(g2w_ref[...], gg, preferred_element_type=jnp.float32) + g2b_ref[...])

    @pl.when(pl.program_id(0) == 0)
    def _():
        gsum_ref[...] = jnp.zeros_like(gsum_ref)

    gsum_ref[...] += jnp.sum(gv)


STRIP = 8


def _shifts_only(src_ref, m1_ref, p1_ref):
    c, hh, ww = src_ref.shape
    f32 = jnp.float32
    z1 = jnp.zeros((c, 1, ww), f32)
    m1_ref[:, 1:hh, :] = src_ref[:, 0:hh - 1, :].astype(f32)
    m1_ref[:, 0:1, :] = z1
    p1_ref[:, 0:hh - 1, :] = src_ref[:, 1:hh, :].astype(f32)
    p1_ref[:, hh - 1:hh, :] = z1


def _build_shifts(in_ref, cen_ref, m1_ref, p1_ref):
    cen_ref[...] = in_ref[...].astype(jnp.float32)
    _shifts_only(in_ref, m1_ref, p1_ref)


def _dw_strips(cen_ref, m1_ref, p1_ref, ws):
    c, hh, ww = cen_ref.shape
    f32 = jnp.float32
    z = jnp.zeros((STRIP, 1), f32)
    for s in range(hh // STRIP):
        r0 = s * STRIP
        a0 = m1_ref[:, r0:r0 + STRIP, :]
        a1 = cen_ref[:, r0:r0 + STRIP, :].astype(f32)
        a2 = p1_ref[:, r0:r0 + STRIP, :]
        for ci in range(c):
            w0, w1, w2 = a0[ci], a1[ci], a2[ci]
            t = ws[ci]
            y0 = w0 * t[0] + w1 * t[3] + w2 * t[6]
            y1 = w0 * t[1] + w1 * t[4] + w2 * t[7]
            y2 = w0 * t[2] + w1 * t[5] + w2 * t[8]
            left = jnp.concatenate([y2[:, 1:], z], axis=1)
            right = jnp.concatenate([z, y0[:, :-1]], axis=1)
            yield ci, r0, y1 + left + right


def _smem_taps(w_ref, c):
    return [[w_ref[0, ci, t] for t in range(9)] for ci in range(c)]


def _k2(x_ref, w_ref, o_ref, cen_ref, m1_ref, p1_ref):
    ws = _smem_taps(w_ref, x_ref.shape[0])
    _build_shifts(x_ref, cen_ref, m1_ref, p1_ref)
    for ci, r0, acc in _dw_strips(cen_ref, m1_ref, p1_ref, ws):
        o_ref[ci, r0:r0 + STRIP, :] = acc.astype(o_ref.dtype)


def _k3(q_ref, k_ref, f_ref, st_ref):
    q = q_ref[...]
    k = k_ref[...]

    @pl.when(pl.program_id(0) == 0)
    def _():
        f_ref[...] = jnp.zeros_like(f_ref)
        st_ref[...] = jnp.zeros_like(st_ref)

    f_ref[...] += jax.lax.dot_general(
        q, k, (((1,), (1,)), ((), ())), preferred_element_type=jnp.float32)
    st_ref[:, 0:1] += jnp.sum(q * q, axis=1, keepdims=True)
    st_ref[:, 1:2] += jnp.sum(k * k, axis=1, keepdims=True)


def _k4(f_ref, st_ref, stt_ref, tv_ref, gs_ref, a_ref):
    dkf = jnp.clip(jnp.floor(CH * gs_ref[0, 0] / N), 1.0, float(CH))

    qn = jnp.maximum(jnp.sqrt(st_ref[:, 0:1]), EPS_L2)
    knt = jnp.maximum(jnp.sqrt(stt_ref[1:2, :]), EPS_L2)
    fn = f_ref[...] / qn / knt * tv_ref[...]

    rows = jax.lax.broadcasted_iota(jnp.int32, (DIM, CH), 0)
    a = jnp.zeros((DIM, CH), jnp.float32)
    for h in range(HEADS):
        in_h = (rows // CH) == h
        a = jnp.where(in_h, fn[:, h * CH:(h + 1) * CH], a)

    iota = jax.lax.broadcasted_iota(jnp.int32, (DIM, CH), 1)
    w = a
    keep = jnp.zeros((DIM, CH), jnp.bool_)
    neg = jnp.float32(-jnp.inf)
    for it in range(CH):
        m = jnp.max(w, axis=1, keepdims=True)
        eq = w == m
        midx = jnp.min(jnp.where(eq, iota, jnp.int32(CH)), axis=1, keepdims=True)
        first = iota == midx
        keep = keep | (first & (jnp.float32(it) < dkf))
        w = jnp.where(first, neg, w)

    s = jnp.where(keep, a, neg)
    mx = jnp.max(s, axis=1, keepdims=True)
    e = jnp.exp(s - mx)
    p = e / jnp.sum(e, axis=1, keepdims=True)

    a_ref[...] = jnp.zeros_like(a_ref)
    for h in range(HEADS):
        a_ref[h * CH:(h + 1) * CH, h * CH:(h + 1) * CH] = p[h * CH:(h + 1) * CH, :]


def _k5(x_ref, v_ref, abd_ref, pow_ref, nw_ref, nb_ref, piw_ref,
        x1_ref, h0_ref):
    av = jnp.dot(abd_ref[...].astype(jnp.bfloat16), v_ref[...],
                 preferred_element_type=jnp.float32)
    x1 = x_ref[...] + jnp.dot(pow_ref[...], av, preferred_element_type=jnp.float32)
    x1_ref[...] = x1
    xn1 = _ln(x1, nw_ref[...], nb_ref[...])
    h0_ref[...] = jnp.dot(piw_ref[...], xn1.astype(jnp.bfloat16),
                          preferred_element_type=jnp.float32).astype(jnp.bfloat16)


def _k6(a_ref, b_ref, wa_ref, wb_ref, w1_ref, w2_ref, m_ref,
        cen_ref, m1_ref, p1_ref, sa_ref, sb_ref, n1_ref, q1_ref):
    c = a_ref.shape[0]
    wsa = _smem_taps(wa_ref, c)
    _build_shifts(a_ref, cen_ref, m1_ref, p1_ref)
    for ci, r0, acc in _dw_strips(cen_ref, m1_ref, p1_ref, wsa):
        sa_ref[ci, r0:r0 + STRIP, :] = acc
    wsb = _smem_taps(wb_ref, c)
    _build_shifts(b_ref, cen_ref, m1_ref, p1_ref)
    for ci, r0, acc in _dw_strips(cen_ref, m1_ref, p1_ref, wsb):
        sb_ref[ci, r0:r0 + STRIP, :] = acc
    ws1 = _smem_taps(w1_ref, c)
    ws2 = _smem_taps(w2_ref, c)
    _shifts_only(sa_ref, m1_ref, p1_ref)
    _shifts_only(sb_ref, n1_ref, q1_ref)
    for (ci, r0, t1), (_, _, t2) in zip(
            _dw_strips(sa_ref, m1_ref, p1_ref, ws1),
            _dw_strips(sb_ref, n1_ref, q1_ref, ws2)):
        v1 = jnp.tanh(t1) + sa_ref[ci, r0:r0 + STRIP, :]
        v2 = jnp.tanh(t2) + sb_ref[ci, r0:r0 + STRIP, :]
        m_ref[ci, r0:r0 + STRIP, :] = (v1 * v2).astype(m_ref.dtype)


def _k7(x1_ref, m_ref, ipow_ref, o_ref):
    o_ref[...] = x1_ref[...] + jnp.dot(
        ipow_ref[...], m_ref[...], preferred_element_type=jnp.float32)


def kernel(x, norm_w, norm_b, temp, q_w, q_dw, kv_w, kv_dw, po_w, g1_w, g1_b,
           g2_w, g2_b, pi_w, dw_w, dw1_w, dw2_w, ipo_w):
    f32 = jnp.float32
    x2d = x.reshape(DIM, N)
    nw = norm_w.reshape(DIM, 1)
    nb = norm_b.reshape(DIM, 1)
    kvw = kv_w.reshape(2 * DIM, DIM)
    wqk = jnp.concatenate([q_w.reshape(DIM, DIM), kvw[:DIM]], axis=0)
    wv = kvw[DIM:]
    g1w = g1_w.reshape(DIM // 2, DIM)
    g1b = g1_b.reshape(DIM // 2, 1)
    g2w = g2_w.reshape(1, DIM // 2)
    g2b = g2_b.reshape(1, 1)
    kvdw = kv_dw.reshape(2 * DIM, 9)
    wdw_qk = jnp.concatenate([q_dw.reshape(DIM, 9), kvdw[:DIM]],
                             axis=0).reshape(2 * DIM // CT2, CT2, 9)
    wdw_v = kvdw[DIM:].reshape(DIM // CT2, CT2, 9)
    tvec = jnp.repeat(temp.reshape(HEADS), CH).reshape(DIM, 1)
    pow_ = po_w.reshape(DIM, DIM)
    piw = pi_w.reshape(2 * HIDDEN, DIM).astype(jnp.bfloat16)
    dwa = dw_w.reshape(2 * HIDDEN, 9)[:HIDDEN].reshape(HIDDEN // CT6, CT6, 9)
    dwb = dw_w.reshape(2 * HIDDEN, 9)[HIDDEN:].reshape(HIDDEN // CT6, CT6, 9)
    dw1 = dw1_w.reshape(HIDDEN // CT6, CT6, 9)
    dw2 = dw2_w.reshape(HIDDEN // CT6, CT6, 9)
    ipow = ipo_w.reshape(DIM, HIDDEN).astype(jnp.bfloat16)

    qk0, v0, gsum = pl.pallas_call(
        _k1,
        grid=(GN,),
        in_specs=[
            pl.BlockSpec((DIM, NT), lambda i: (0, i)),
            pl.BlockSpec((DIM, 1), lambda i: (0, 0)),
            pl.BlockSpec((DIM, 1), lambda i: (0, 0)),
            pl.BlockSpec((2 * DIM, DIM), lambda i: (0, 0)),
            pl.BlockSpec((DIM, DIM), lambda i: (0, 0)),
            pl.BlockSpec((DIM // 2, DIM), lambda i: (0, 0)),
            pl.BlockSpec((DIM // 2, 1), lambda i: (0, 0)),
            pl.BlockSpec((1, DIM // 2), lambda i: (0, 0)),
            pl.BlockSpec((1, 1), lambda i: (0, 0)),
        ],
        out_specs=[
            pl.BlockSpec((2 * DIM, NT), lambda i: (0, i)),
            pl.BlockSpec((DIM, NT), lambda i: (0, i)),
            pl.BlockSpec((8, 128), lambda i: (0, 0)),
        ],
        out_shape=[
            jax.ShapeDtypeStruct((2 * DIM, N), f32),
            jax.ShapeDtypeStruct((DIM, N), jnp.bfloat16),
            jax.ShapeDtypeStruct((8, 128), f32),
        ],
        compiler_params=_ARB,
    )(x2d, nw, nb, wqk, wv, g1w, g1b, g2w, g2b)

    qk = pl.pallas_call(
        _k2,
        grid=(2 * DIM // CT2,),
        in_specs=[
            pl.BlockSpec((CT2, HW, HW), lambda c: (c, 0, 0)),
            pl.BlockSpec((1, CT2, 9), lambda c: (c, 0, 0),
                         memory_space=pltpu.SMEM),
        ],
        out_specs=pl.BlockSpec((CT2, HW, HW), lambda c: (c, 0, 0)),
        out_shape=jax.ShapeDtypeStruct((2 * DIM, HW, HW), f32),
        scratch_shapes=[pltpu.VMEM((CT2, HW, HW), f32)] * 3,
        compiler_params=_PAR,
    )(qk0.reshape(2 * DIM, HW, HW), wdw_qk)
    qk2d = qk.reshape(2 * DIM, N)

    v = pl.pallas_call(
        _k2,
        grid=(DIM // CT2,),
        in_specs=[
            pl.BlockSpec((CT2, HW, HW), lambda c: (c, 0, 0)),
            pl.BlockSpec((1, CT2, 9), lambda c: (c, 0, 0),
                         memory_space=pltpu.SMEM),
        ],
        out_specs=pl.BlockSpec((CT2, HW, HW), lambda c: (c, 0, 0)),
        out_shape=jax.ShapeDtypeStruct((DIM, HW, HW), jnp.bfloat16),
        scratch_shapes=[pltpu.VMEM((CT2, HW, HW), f32)] * 3,
        compiler_params=_PAR,
    )(v0.reshape(DIM, HW, HW), wdw_v)
    v2d = v.reshape(DIM, N)

    gram, stats = pl.pallas_call(
        _k3,
        grid=(GN,),
        in_specs=[
            pl.BlockSpec((DIM, NT), lambda i: (0, i)),
            pl.BlockSpec((DIM, NT), lambda i: (1, i)),
        ],
        out_specs=[
            pl.BlockSpec((DIM, DIM), lambda i: (0, 0)),
            pl.BlockSpec((DIM, 128), lambda i: (0, 0)),
        ],
        out_shape=[
            jax.ShapeDtypeStruct((DIM, DIM), f32),
            jax.ShapeDtypeStruct((DIM, 128), f32),
        ],
        compiler_params=_ARB,
    )(qk2d, qk2d)

    abd = pl.pallas_call(
        _k4,
        out_shape=jax.ShapeDtypeStruct((DIM, DIM), f32),
    )(gram, stats, stats.T, tvec, gsum)

    x1, h0 = pl.pallas_call(
        _k5,
        grid=(GN,),
        in_specs=[
            pl.BlockSpec((DIM, NT), lambda i: (0, i)),
            pl.BlockSpec((DIM, NT), lambda i: (0, i)),
            pl.BlockSpec((DIM, DIM), lambda i: (0, 0)),
            pl.BlockSpec((DIM, DIM), lambda i: (0, 0)),
            pl.BlockSpec((DIM, 1), lambda i: (0, 0)),
            pl.BlockSpec((DIM, 1), lambda i: (0, 0)),
            pl.BlockSpec((2 * HIDDEN, DIM), lambda i: (0, 0)),
        ],
        out_specs=[
            pl.BlockSpec((DIM, NT), lambda i: (0, i)),
            pl.BlockSpec((2 * HIDDEN, NT), lambda i: (0, i)),
        ],
        out_shape=[
            jax.ShapeDtypeStruct((DIM, N), f32),
            jax.ShapeDtypeStruct((2 * HIDDEN, N), jnp.bfloat16),
        ],
        compiler_params=_PAR,
    )(x2d, v2d, abd, pow_, nw, nb, piw)

    m = pl.pallas_call(
        _k6,
        grid=(HIDDEN // CT6,),
        in_specs=[
            pl.BlockSpec((CT6, HW, HW), lambda c: (c, 0, 0)),
            pl.BlockSpec((CT6, HW, HW), lambda c: (c + HIDDEN // CT6, 0, 0)),
            pl.BlockSpec((1, CT6, 9), lambda c: (c, 0, 0),
                         memory_space=pltpu.SMEM),
            pl.BlockSpec((1, CT6, 9), lambda c: (c, 0, 0),
                         memory_space=pltpu.SMEM),
            pl.BlockSpec((1, CT6, 9), lambda c: (c, 0, 0),
                         memory_space=pltpu.SMEM),
            pl.BlockSpec((1, CT6, 9), lambda c: (c, 0, 0),
                         memory_space=pltpu.SMEM),
        ],
        out_specs=pl.BlockSpec((CT6, HW, HW), lambda c: (c, 0, 0)),
        out_shape=jax.ShapeDtypeStruct((HIDDEN, HW, HW), jnp.bfloat16),
        scratch_shapes=[pltpu.VMEM((CT6, HW, HW), f32)] * 7,
        compiler_params=_PAR,
    )(h0.reshape(2 * HIDDEN, HW, HW), h0.reshape(2 * HIDDEN, HW, HW),
      dwa, dwb, dw1, dw2)

    out = pl.pallas_call(
        _k7,
        grid=(GN,),
        in_specs=[
            pl.BlockSpec((DIM, NT), lambda i: (0, i)),
            pl.BlockSpec((HIDDEN, NT), lambda i: (0, i)),
            pl.BlockSpec((DIM, HIDDEN), lambda i: (0, 0)),
        ],
        out_specs=pl.BlockSpec((DIM, NT), lambda i: (0, i)),
        out_shape=jax.ShapeDtypeStruct((DIM, N), f32),
        compiler_params=_PAR,
    )(x1, m.reshape(HIDDEN, N), ipow)

    return out.reshape(1, DIM, HW, HW)


# speedup vs baseline: 1.6119x; 1.0514x over previous
import jax
import jax.numpy as jnp
from jax.experimental import pallas as pl
from jax.experimental.pallas import tpu as pltpu

DIM = 192
HEADS = 8
CH = DIM // HEADS
HIDDEN = int(DIM * 2.66)
HW = 224
N = HW * HW
NT = 1024
GN = N // NT
EPS_LN = 1e-6
EPS_L2 = 1e-12

CT2 = 16
CT6 = 10

_PAR = pltpu.CompilerParams(dimension_semantics=("parallel",))
_ARB = pltpu.CompilerParams(dimension_semantics=("arbitrary",))


def _ln(x, w, b):
    u = jnp.mean(x, axis=0, keepdims=True)
    s = jnp.mean((x - u) * (x - u), axis=0, keepdims=True)
    return w * ((x - u) * jax.lax.rsqrt(s + EPS_LN)) + b


def _k1(x_ref, nw_ref, nb_ref, wqk_ref, wv_ref, g1w_ref, g1b_ref, g2w_ref, g2b_ref,
        qk0_ref, v0_ref, gsum_ref):
    xn = _ln(x_ref[...], nw_ref[...], nb_ref[...])
    qk0_ref[...] = jnp.dot(wqk_ref[...], xn,
                           preferred_element_type=jnp.float32).astype(jnp.bfloat16)
    v0_ref[...] = jnp.dot(wv_ref[...], xn,
                          preferred_element_type=jnp.float32).astype(jnp.bfloat16)
    gg = jnp.maximum(
        jnp.dot(g1w_ref[...], xn, preferred_element_type=jnp.float32) + g1b_ref[...], 0.0)
    gv = jax.nn.sigmoid(
        jnp.dot(g2w_ref[...], gg, preferred_element_type=jnp.float32) + g2b_ref[...])

    @pl.when(pl.program_id(0) == 0)
    def _():
        gsum_ref[...] = jnp.zeros_like(gsum_ref)

    gsum_ref[...] += jnp.sum(gv)


STRIP = 8


def _shifts_only(src_ref, m1_ref, p1_ref):
    c, hh, ww = src_ref.shape
    f32 = jnp.float32
    z1 = jnp.zeros((c, 1, ww), f32)
    m1_ref[:, 1:hh, :] = src_ref[:, 0:hh - 1, :].astype(f32)
    m1_ref[:, 0:1, :] = z1
    p1_ref[:, 0:hh - 1, :] = src_ref[:, 1:hh, :].astype(f32)
    p1_ref[:, hh - 1:hh, :] = z1


def _build_shifts(in_ref, cen_ref, m1_ref, p1_ref):
    cen_ref[...] = in_ref[...].astype(jnp.float32)
    _shifts_only(in_ref, m1_ref, p1_ref)


def _dw_strips(cen_ref, m1_ref, p1_ref, ws):
    c, hh, ww = cen_ref.shape
    f32 = jnp.float32
    z = jnp.zeros((STRIP, 1), f32)
    for s in range(hh // STRIP):
        r0 = s * STRIP
        a0 = m1_ref[:, r0:r0 + STRIP, :]
        a1 = cen_ref[:, r0:r0 + STRIP, :].astype(f32)
        a2 = p1_ref[:, r0:r0 + STRIP, :]
        for ci in range(c):
            w0, w1, w2 = a0[ci], a1[ci], a2[ci]
            t = ws[ci]
            y0 = w0 * t[0] + w1 * t[3] + w2 * t[6]
            y1 = w0 * t[1] + w1 * t[4] + w2 * t[7]
            y2 = w0 * t[2] + w1 * t[5] + w2 * t[8]
            left = jnp.concatenate([y2[:, 1:], z], axis=1)
            right = jnp.concatenate([z, y0[:, :-1]], axis=1)
            yield ci, r0, y1 + left + right


def _smem_taps(w_ref, c):
    return [[w_ref[0, ci, t] for t in range(9)] for ci in range(c)]


def _k2(x_ref, w_ref, o_ref, cen_ref, m1_ref, p1_ref):
    ws = _smem_taps(w_ref, x_ref.shape[0])
    _build_shifts(x_ref, cen_ref, m1_ref, p1_ref)
    for ci, r0, acc in _dw_strips(cen_ref, m1_ref, p1_ref, ws):
        o_ref[ci, r0:r0 + STRIP, :] = acc.astype(o_ref.dtype)


def _k3(q_ref, k_ref, f_ref, st_ref):
    q = q_ref[...]
    k = k_ref[...]
    q32 = q.astype(jnp.float32)
    k32 = k.astype(jnp.float32)

    @pl.when(pl.program_id(0) == 0)
    def _():
        f_ref[...] = jnp.zeros_like(f_ref)
        st_ref[...] = jnp.zeros_like(st_ref)

    f_ref[...] += jax.lax.dot_general(
        q, k, (((1,), (1,)), ((), ())), preferred_element_type=jnp.float32)
    st_ref[:, 0:1] += jnp.sum(q32 * q32, axis=1, keepdims=True)
    st_ref[:, 1:2] += jnp.sum(k32 * k32, axis=1, keepdims=True)


def _k4(f_ref, st_ref, stt_ref, tv_ref, gs_ref, a_ref):
    dkf = jnp.clip(jnp.floor(CH * gs_ref[0, 0] / N), 1.0, float(CH))

    qn = jnp.maximum(jnp.sqrt(st_ref[:, 0:1]), EPS_L2)
    knt = jnp.maximum(jnp.sqrt(stt_ref[1:2, :]), EPS_L2)
    fn = f_ref[...] / qn / knt * tv_ref[...]

    rows = jax.lax.broadcasted_iota(jnp.int32, (DIM, CH), 0)
    a = jnp.zeros((DIM, CH), jnp.float32)
    for h in range(HEADS):
        in_h = (rows // CH) == h
        a = jnp.where(in_h, fn[:, h * CH:(h + 1) * CH], a)

    iota = jax.lax.broadcasted_iota(jnp.int32, (DIM, CH), 1)
    w = a
    keep = jnp.zeros((DIM, CH), jnp.bool_)
    neg = jnp.float32(-jnp.inf)
    for it in range(CH):
        m = jnp.max(w, axis=1, keepdims=True)
        eq = w == m
        midx = jnp.min(jnp.where(eq, iota, jnp.int32(CH)), axis=1, keepdims=True)
        first = iota == midx
        keep = keep | (first & (jnp.float32(it) < dkf))
        w = jnp.where(first, neg, w)

    s = jnp.where(keep, a, neg)
    mx = jnp.max(s, axis=1, keepdims=True)
    e = jnp.exp(s - mx)
    p = e / jnp.sum(e, axis=1, keepdims=True)

    a_ref[...] = jnp.zeros_like(a_ref)
    for h in range(HEADS):
        a_ref[h * CH:(h + 1) * CH, h * CH:(h + 1) * CH] = p[h * CH:(h + 1) * CH, :]


def _k5(x_ref, v_ref, abd_ref, pow_ref, nw_ref, nb_ref, piw_ref,
        x1_ref, h0_ref):
    av = jnp.dot(abd_ref[...].astype(jnp.bfloat16), v_ref[...],
                 preferred_element_type=jnp.float32)
    x1 = x_ref[...] + jnp.dot(pow_ref[...], av, preferred_element_type=jnp.float32)
    x1_ref[...] = x1
    xn1 = _ln(x1, nw_ref[...], nb_ref[...])
    h0_ref[...] = jnp.dot(piw_ref[...], xn1.astype(jnp.bfloat16),
                          preferred_element_type=jnp.float32).astype(jnp.bfloat16)


def _k6(a_ref, b_ref, wa_ref, wb_ref, w1_ref, w2_ref, m_ref,
        cen_ref, m1_ref, p1_ref, sa_ref, sb_ref, n1_ref, q1_ref):
    c = a_ref.shape[0]
    wsa = _smem_taps(wa_ref, c)
    _build_shifts(a_ref, cen_ref, m1_ref, p1_ref)
    for ci, r0, acc in _dw_strips(cen_ref, m1_ref, p1_ref, wsa):
        sa_ref[ci, r0:r0 + STRIP, :] = acc
    wsb = _smem_taps(wb_ref, c)
    _build_shifts(b_ref, cen_ref, m1_ref, p1_ref)
    for ci, r0, acc in _dw_strips(cen_ref, m1_ref, p1_ref, wsb):
        sb_ref[ci, r0:r0 + STRIP, :] = acc
    ws1 = _smem_taps(w1_ref, c)
    ws2 = _smem_taps(w2_ref, c)
    _shifts_only(sa_ref, m1_ref, p1_ref)
    _shifts_only(sb_ref, n1_ref, q1_ref)
    for (ci, r0, t1), (_, _, t2) in zip(
            _dw_strips(sa_ref, m1_ref, p1_ref, ws1),
            _dw_strips(sb_ref, n1_ref, q1_ref, ws2)):
        v1 = jnp.tanh(t1) + sa_ref[ci, r0:r0 + STRIP, :]
        v2 = jnp.tanh(t2) + sb_ref[ci, r0:r0 + STRIP, :]
        m_ref[ci, r0:r0 + STRIP, :] = (v1 * v2).astype(m_ref.dtype)


def _k7(x1_ref, m_ref, ipow_ref, o_ref):
    o_ref[...] = x1_ref[...] + jnp.dot(
        ipow_ref[...], m_ref[...], preferred_element_type=jnp.float32)


def kernel(x, norm_w, norm_b, temp, q_w, q_dw, kv_w, kv_dw, po_w, g1_w, g1_b,
           g2_w, g2_b, pi_w, dw_w, dw1_w, dw2_w, ipo_w):
    f32 = jnp.float32
    x2d = x.reshape(DIM, N)
    nw = norm_w.reshape(DIM, 1)
    nb = norm_b.reshape(DIM, 1)
    kvw = kv_w.reshape(2 * DIM, DIM)
    wqk = jnp.concatenate([q_w.reshape(DIM, DIM), kvw[:DIM]], axis=0)
    wv = kvw[DIM:]
    g1w = g1_w.reshape(DIM // 2, DIM)
    g1b = g1_b.reshape(DIM // 2, 1)
    g2w = g2_w.reshape(1, DIM // 2)
    g2b = g2_b.reshape(1, 1)
    kvdw = kv_dw.reshape(2 * DIM, 9)
    wdw_qk = jnp.concatenate([q_dw.reshape(DIM, 9), kvdw[:DIM]],
                             axis=0).reshape(2 * DIM // CT2, CT2, 9)
    wdw_v = kvdw[DIM:].reshape(DIM // CT2, CT2, 9)
    tvec = jnp.repeat(temp.reshape(HEADS), CH).reshape(DIM, 1)
    pow_ = po_w.reshape(DIM, DIM)
    piw = pi_w.reshape(2 * HIDDEN, DIM).astype(jnp.bfloat16)
    dwa = dw_w.reshape(2 * HIDDEN, 9)[:HIDDEN].reshape(HIDDEN // CT6, CT6, 9)
    dwb = dw_w.reshape(2 * HIDDEN, 9)[HIDDEN:].reshape(HIDDEN // CT6, CT6, 9)
    dw1 = dw1_w.reshape(HIDDEN // CT6, CT6, 9)
    dw2 = dw2_w.reshape(HIDDEN // CT6, CT6, 9)
    ipow = ipo_w.reshape(DIM, HIDDEN).astype(jnp.bfloat16)

    qk0, v0, gsum = pl.pallas_call(
        _k1,
        grid=(GN,),
        in_specs=[
            pl.BlockSpec((DIM, NT), lambda i: (0, i)),
            pl.BlockSpec((DIM, 1), lambda i: (0, 0)),
            pl.BlockSpec((DIM, 1), lambda i: (0, 0)),
            pl.BlockSpec((2 * DIM, DIM), lambda i: (0, 0)),
            pl.BlockSpec((DIM, DIM), lambda i: (0, 0)),
            pl.BlockSpec((DIM // 2, DIM), lambda i: (0, 0)),
            pl.BlockSpec((DIM // 2, 1), lambda i: (0, 0)),
            pl.BlockSpec((1, DIM // 2), lambda i: (0, 0)),
            pl.BlockSpec((1, 1), lambda i: (0, 0)),
        ],
        out_specs=[
            pl.BlockSpec((2 * DIM, NT), lambda i: (0, i)),
            pl.BlockSpec((DIM, NT), lambda i: (0, i)),
            pl.BlockSpec((8, 128), lambda i: (0, 0)),
        ],
        out_shape=[
            jax.ShapeDtypeStruct((2 * DIM, N), jnp.bfloat16),
            jax.ShapeDtypeStruct((DIM, N), jnp.bfloat16),
            jax.ShapeDtypeStruct((8, 128), f32),
        ],
        compiler_params=_ARB,
    )(x2d, nw, nb, wqk, wv, g1w, g1b, g2w, g2b)

    qk = pl.pallas_call(
        _k2,
        grid=(2 * DIM // CT2,),
        in_specs=[
            pl.BlockSpec((CT2, HW, HW), lambda c: (c, 0, 0)),
            pl.BlockSpec((1, CT2, 9), lambda c: (c, 0, 0),
                         memory_space=pltpu.SMEM),
        ],
        out_specs=pl.BlockSpec((CT2, HW, HW), lambda c: (c, 0, 0)),
        out_shape=jax.ShapeDtypeStruct((2 * DIM, HW, HW), jnp.bfloat16),
        scratch_shapes=[pltpu.VMEM((CT2, HW, HW), f32)] * 3,
        compiler_params=_PAR,
    )(qk0.reshape(2 * DIM, HW, HW), wdw_qk)
    qk2d = qk.reshape(2 * DIM, N)

    v = pl.pallas_call(
        _k2,
        grid=(DIM // CT2,),
        in_specs=[
            pl.BlockSpec((CT2, HW, HW), lambda c: (c, 0, 0)),
            pl.BlockSpec((1, CT2, 9), lambda c: (c, 0, 0),
                         memory_space=pltpu.SMEM),
        ],
        out_specs=pl.BlockSpec((CT2, HW, HW), lambda c: (c, 0, 0)),
        out_shape=jax.ShapeDtypeStruct((DIM, HW, HW), jnp.bfloat16),
        scratch_shapes=[pltpu.VMEM((CT2, HW, HW), f32)] * 3,
        compiler_params=_PAR,
    )(v0.reshape(DIM, HW, HW), wdw_v)
    v2d = v.reshape(DIM, N)

    gram, stats = pl.pallas_call(
        _k3,
        grid=(GN,),
        in_specs=[
            pl.BlockSpec((DIM, NT), lambda i: (0, i)),
            pl.BlockSpec((DIM, NT), lambda i: (1, i)),
        ],
        out_specs=[
            pl.BlockSpec((DIM, DIM), lambda i: (0, 0)),
            pl.BlockSpec((DIM, 128), lambda i: (0, 0)),
        ],
        out_shape=[
            jax.ShapeDtypeStruct((DIM, DIM), f32),
            jax.ShapeDtypeStruct((DIM, 128), f32),
        ],
        compiler_params=_ARB,
    )(qk2d, qk2d)

    abd = pl.pallas_call(
        _k4,
        out_shape=jax.ShapeDtypeStruct((DIM, DIM), f32),
    )(gram, stats, stats.T, tvec, gsum)

    x1, h0 = pl.pallas_call(
        _k5,
        grid=(GN,),
        in_specs=[
            pl.BlockSpec((DIM, NT), lambda i: (0, i)),
            pl.BlockSpec((DIM, NT), lambda i: (0, i)),
            pl.BlockSpec((DIM, DIM), lambda i: (0, 0)),
            pl.BlockSpec((DIM, DIM), lambda i: (0, 0)),
            pl.BlockSpec((DIM, 1), lambda i: (0, 0)),
            pl.BlockSpec((DIM, 1), lambda i: (0, 0)),
            pl.BlockSpec((2 * HIDDEN, DIM), lambda i: (0, 0)),
        ],
        out_specs=[
            pl.BlockSpec((DIM, NT), lambda i: (0, i)),
            pl.BlockSpec((2 * HIDDEN, NT), lambda i: (0, i)),
        ],
        out_shape=[
            jax.ShapeDtypeStruct((DIM, N), f32),
            jax.ShapeDtypeStruct((2 * HIDDEN, N), jnp.bfloat16),
        ],
        compiler_params=_PAR,
    )(x2d, v2d, abd, pow_, nw, nb, piw)

    m = pl.pallas_call(
        _k6,
        grid=(HIDDEN // CT6,),
        in_specs=[
            pl.BlockSpec((CT6, HW, HW), lambda c: (c, 0, 0)),
            pl.BlockSpec((CT6, HW, HW), lambda c: (c + HIDDEN // CT6, 0, 0)),
            pl.BlockSpec((1, CT6, 9), lambda c: (c, 0, 0),
                         memory_space=pltpu.SMEM),
            pl.BlockSpec((1, CT6, 9), lambda c: (c, 0, 0),
                         memory_space=pltpu.SMEM),
            pl.BlockSpec((1, CT6, 9), lambda c: (c, 0, 0),
                         memory_space=pltpu.SMEM),
            pl.BlockSpec((1, CT6, 9), lambda c: (c, 0, 0),
                         memory_space=pltpu.SMEM),
        ],
        out_specs=pl.BlockSpec((CT6, HW, HW), lambda c: (c, 0, 0)),
        out_shape=jax.ShapeDtypeStruct((HIDDEN, HW, HW), jnp.bfloat16),
        scratch_shapes=[pltpu.VMEM((CT6, HW, HW), f32)] * 7,
        compiler_params=_PAR,
    )(h0.reshape(2 * HIDDEN, HW, HW), h0.reshape(2 * HIDDEN, HW, HW),
      dwa, dwb, dw1, dw2)

    out = pl.pallas_call(
        _k7,
        grid=(GN,),
        in_specs=[
            pl.BlockSpec((DIM, NT), lambda i: (0, i)),
            pl.BlockSpec((HIDDEN, NT), lambda i: (0, i)),
            pl.BlockSpec((DIM, HIDDEN), lambda i: (0, 0)),
        ],
        out_specs=pl.BlockSpec((DIM, NT), lambda i: (0, i)),
        out_shape=jax.ShapeDtypeStruct((DIM, N), f32),
        compiler_params=_PAR,
    )(x1, m.reshape(HIDDEN, N), ipow)

    return out.reshape(1, DIM, HW, HW)


# speedup vs baseline: 1.6882x; 1.0473x over previous
import jax
import jax.numpy as jnp
from jax.experimental import pallas as pl
from jax.experimental.pallas import tpu as pltpu

DIM = 192
HEADS = 8
CH = DIM // HEADS
HIDDEN = int(DIM * 2.66)
HW = 224
N = HW * HW
NT = 1024
GN = N // NT
EPS_LN = 1e-6
EPS_L2 = 1e-12

CT2 = 16
CT6 = 10

_PAR = pltpu.CompilerParams(dimension_semantics=("parallel",))
_ARB = pltpu.CompilerParams(dimension_semantics=("arbitrary",))


def _ln(x, w, b):
    u = jnp.mean(x, axis=0, keepdims=True)
    s = jnp.mean((x - u) * (x - u), axis=0, keepdims=True)
    return w * ((x - u) * jax.lax.rsqrt(s + EPS_LN)) + b


def _k1(x_ref, nw_ref, nb_ref, wqkv_ref, g1w_ref, g1b_ref, g2w_ref, g2b_ref,
        qkv0_ref, gsum_ref):
    xn = _ln(x_ref[...], nw_ref[...], nb_ref[...])
    qkv0_ref[...] = jnp.dot(wqkv_ref[...], xn,
                            preferred_element_type=jnp.float32).astype(jnp.bfloat16)
    gg = jnp.maximum(
        jnp.dot(g1w_ref[...], xn, preferred_element_type=jnp.float32) + g1b_ref[...], 0.0)
    gv = jax.nn.sigmoid(
        jnp.dot(g2w_ref[...], gg, preferred_element_type=jnp.float32) + g2b_ref[...])

    @pl.when(pl.program_id(0) == 0)
    def _():
        gsum_ref[...] = jnp.zeros_like(gsum_ref)

    gsum_ref[...] += jnp.sum(gv)


STRIP = 8


def _shifts_only(src_ref, m1_ref, p1_ref):
    c, hh, ww = src_ref.shape
    f32 = jnp.float32
    z1 = jnp.zeros((c, 1, ww), f32)
    m1_ref[:, 1:hh, :] = src_ref[:, 0:hh - 1, :].astype(f32)
    m1_ref[:, 0:1, :] = z1
    p1_ref[:, 0:hh - 1, :] = src_ref[:, 1:hh, :].astype(f32)
    p1_ref[:, hh - 1:hh, :] = z1


def _build_shifts(in_ref, cen_ref, m1_ref, p1_ref):
    cen_ref[...] = in_ref[...].astype(jnp.float32)
    _shifts_only(in_ref, m1_ref, p1_ref)


def _dw_strips(cen_ref, m1_ref, p1_ref, ws):
    c, hh, ww = cen_ref.shape
    f32 = jnp.float32
    z = jnp.zeros((STRIP, 1), f32)
    for s in range(hh // STRIP):
        r0 = s * STRIP
        a0 = m1_ref[:, r0:r0 + STRIP, :]
        a1 = cen_ref[:, r0:r0 + STRIP, :].astype(f32)
        a2 = p1_ref[:, r0:r0 + STRIP, :]
        for ci in range(c):
            w0, w1, w2 = a0[ci], a1[ci], a2[ci]
            t = ws[ci]
            y0 = w0 * t[0] + w1 * t[3] + w2 * t[6]
            y1 = w0 * t[1] + w1 * t[4] + w2 * t[7]
            y2 = w0 * t[2] + w1 * t[5] + w2 * t[8]
            left = jnp.concatenate([y2[:, 1:], z], axis=1)
            right = jnp.concatenate([z, y0[:, :-1]], axis=1)
            yield ci, r0, y1 + left + right


def _smem_taps(w_ref, c):
    return [[w_ref[0, ci, t] for t in range(9)] for ci in range(c)]


def _k2(x_ref, w_ref, o_ref, cen_ref, m1_ref, p1_ref):
    ws = _smem_taps(w_ref, x_ref.shape[0])
    _build_shifts(x_ref, cen_ref, m1_ref, p1_ref)
    for ci, r0, acc in _dw_strips(cen_ref, m1_ref, p1_ref, ws):
        o_ref[ci, r0:r0 + STRIP, :] = acc.astype(o_ref.dtype)


def _k3(q_ref, k_ref, f_ref, st_ref):
    q = q_ref[...]
    k = k_ref[...]
    q32 = q.astype(jnp.float32)
    k32 = k.astype(jnp.float32)

    @pl.when(pl.program_id(0) == 0)
    def _():
        f_ref[...] = jnp.zeros_like(f_ref)
        st_ref[...] = jnp.zeros_like(st_ref)

    f_ref[...] += jax.lax.dot_general(
        q, k, (((1,), (1,)), ((), ())), preferred_element_type=jnp.float32)
    st_ref[:, 0:1] += jnp.sum(q32 * q32, axis=1, keepdims=True)
    st_ref[:, 1:2] += jnp.sum(k32 * k32, axis=1, keepdims=True)


def _k4(f_ref, st_ref, stt_ref, tv_ref, gs_ref, a_ref):
    dkf = jnp.clip(jnp.floor(CH * gs_ref[0, 0] / N), 1.0, float(CH))

    qn = jnp.maximum(jnp.sqrt(st_ref[:, 0:1]), EPS_L2)
    knt = jnp.maximum(jnp.sqrt(stt_ref[1:2, :]), EPS_L2)
    fn = f_ref[...] / qn / knt * tv_ref[...]

    rows = jax.lax.broadcasted_iota(jnp.int32, (DIM, CH), 0)
    a = jnp.zeros((DIM, CH), jnp.float32)
    for h in range(HEADS):
        in_h = (rows // CH) == h
        a = jnp.where(in_h, fn[:, h * CH:(h + 1) * CH], a)

    iota = jax.lax.broadcasted_iota(jnp.int32, (DIM, CH), 1)
    w = a
    keep = jnp.zeros((DIM, CH), jnp.bool_)
    neg = jnp.float32(-jnp.inf)
    for it in range(CH):
        m = jnp.max(w, axis=1, keepdims=True)
        eq = w == m
        midx = jnp.min(jnp.where(eq, iota, jnp.int32(CH)), axis=1, keepdims=True)
        first = iota == midx
        keep = keep | (first & (jnp.float32(it) < dkf))
        w = jnp.where(first, neg, w)

    s = jnp.where(keep, a, neg)
    mx = jnp.max(s, axis=1, keepdims=True)
    e = jnp.exp(s - mx)
    p = e / jnp.sum(e, axis=1, keepdims=True)

    a_ref[...] = jnp.zeros_like(a_ref)
    for h in range(HEADS):
        a_ref[h * CH:(h + 1) * CH, h * CH:(h + 1) * CH] = p[h * CH:(h + 1) * CH, :]


def _k5(x_ref, v_ref, abd_ref, pow_ref, nw_ref, nb_ref, piw_ref,
        x1_ref, h0_ref):
    av = jnp.dot(abd_ref[...].astype(jnp.bfloat16), v_ref[...],
                 preferred_element_type=jnp.float32)
    x1 = x_ref[...] + jnp.dot(pow_ref[...], av, preferred_element_type=jnp.float32)
    x1_ref[...] = x1.astype(jnp.bfloat16)
    xn1 = _ln(x1, nw_ref[...], nb_ref[...])
    h0_ref[...] = jnp.dot(piw_ref[...], xn1.astype(jnp.bfloat16),
                          preferred_element_type=jnp.float32).astype(jnp.bfloat16)


def _k6(a_ref, b_ref, wa_ref, wb_ref, w1_ref, w2_ref, m_ref,
        cen_ref, m1_ref, p1_ref, sa_ref, sb_ref, n1_ref, q1_ref):
    c = a_ref.shape[0]
    wsa = _smem_taps(wa_ref, c)
    _build_shifts(a_ref, cen_ref, m1_ref, p1_ref)
    for ci, r0, acc in _dw_strips(cen_ref, m1_ref, p1_ref, wsa):
        sa_ref[ci, r0:r0 + STRIP, :] = acc
    wsb = _smem_taps(wb_ref, c)
    _build_shifts(b_ref, cen_ref, m1_ref, p1_ref)
    for ci, r0, acc in _dw_strips(cen_ref, m1_ref, p1_ref, wsb):
        sb_ref[ci, r0:r0 + STRIP, :] = acc
    ws1 = _smem_taps(w1_ref, c)
    ws2 = _smem_taps(w2_ref, c)
    _shifts_only(sa_ref, m1_ref, p1_ref)
    _shifts_only(sb_ref, n1_ref, q1_ref)
    for (ci, r0, t1), (_, _, t2) in zip(
            _dw_strips(sa_ref, m1_ref, p1_ref, ws1),
            _dw_strips(sb_ref, n1_ref, q1_ref, ws2)):
        v1 = jnp.tanh(t1) + sa_ref[ci, r0:r0 + STRIP, :]
        v2 = jnp.tanh(t2) + sb_ref[ci, r0:r0 + STRIP, :]
        m_ref[ci, r0:r0 + STRIP, :] = (v1 * v2).astype(m_ref.dtype)


def _k7(x1_ref, m_ref, ipow_ref, o_ref):
    o_ref[...] = x1_ref[...].astype(jnp.float32) + jnp.dot(
        ipow_ref[...], m_ref[...], preferred_element_type=jnp.float32)


def kernel(x, norm_w, norm_b, temp, q_w, q_dw, kv_w, kv_dw, po_w, g1_w, g1_b,
           g2_w, g2_b, pi_w, dw_w, dw1_w, dw2_w, ipo_w):
    f32 = jnp.float32
    x2d = x.reshape(DIM, N)
    nw = norm_w.reshape(DIM, 1)
    nb = norm_b.reshape(DIM, 1)
    wqkv = jnp.concatenate([q_w.reshape(DIM, DIM), kv_w.reshape(2 * DIM, DIM)],
                           axis=0)
    g1w = g1_w.reshape(DIM // 2, DIM)
    g1b = g1_b.reshape(DIM // 2, 1)
    g2w = g2_w.reshape(1, DIM // 2)
    g2b = g2_b.reshape(1, 1)
    wdw_qkv = jnp.concatenate([q_dw.reshape(DIM, 9), kv_dw.reshape(2 * DIM, 9)],
                              axis=0).reshape(3 * DIM // CT2, CT2, 9)
    tvec = jnp.repeat(temp.reshape(HEADS), CH).reshape(DIM, 1)
    pow_ = po_w.reshape(DIM, DIM)
    piw = pi_w.reshape(2 * HIDDEN, DIM).astype(jnp.bfloat16)
    dwa = dw_w.reshape(2 * HIDDEN, 9)[:HIDDEN].reshape(HIDDEN // CT6, CT6, 9)
    dwb = dw_w.reshape(2 * HIDDEN, 9)[HIDDEN:].reshape(HIDDEN // CT6, CT6, 9)
    dw1 = dw1_w.reshape(HIDDEN // CT6, CT6, 9)
    dw2 = dw2_w.reshape(HIDDEN // CT6, CT6, 9)
    ipow = ipo_w.reshape(DIM, HIDDEN).astype(jnp.bfloat16)

    qkv0, gsum = pl.pallas_call(
        _k1,
        grid=(GN,),
        in_specs=[
            pl.BlockSpec((DIM, NT), lambda i: (0, i)),
            pl.BlockSpec((DIM, 1), lambda i: (0, 0)),
            pl.BlockSpec((DIM, 1), lambda i: (0, 0)),
            pl.BlockSpec((3 * DIM, DIM), lambda i: (0, 0)),
            pl.BlockSpec((DIM // 2, DIM), lambda i: (0, 0)),
            pl.BlockSpec((DIM // 2, 1), lambda i: (0, 0)),
            pl.BlockSpec((1, DIM // 2), lambda i: (0, 0)),
            pl.BlockSpec((1, 1), lambda i: (0, 0)),
        ],
        out_specs=[
            pl.BlockSpec((3 * DIM, NT), lambda i: (0, i)),
            pl.BlockSpec((8, 128), lambda i: (0, 0)),
        ],
        out_shape=[
            jax.ShapeDtypeStruct((3 * DIM, N), jnp.bfloat16),
            jax.ShapeDtypeStruct((8, 128), f32),
        ],
        compiler_params=_ARB,
    )(x2d, nw, nb, wqkv, g1w, g1b, g2w, g2b)

    qkv = pl.pallas_call(
        _k2,
        grid=(3 * DIM // CT2,),
        in_specs=[
            pl.BlockSpec((CT2, HW, HW), lambda c: (c, 0, 0)),
            pl.BlockSpec((1, CT2, 9), lambda c: (c, 0, 0),
                         memory_space=pltpu.SMEM),
        ],
        out_specs=pl.BlockSpec((CT2, HW, HW), lambda c: (c, 0, 0)),
        out_shape=jax.ShapeDtypeStruct((3 * DIM, HW, HW), jnp.bfloat16),
        scratch_shapes=[pltpu.VMEM((CT2, HW, HW), f32)] * 3,
        compiler_params=_PAR,
    )(qkv0.reshape(3 * DIM, HW, HW), wdw_qkv)
    qkv2d = qkv.reshape(3 * DIM, N)

    gram, stats = pl.pallas_call(
        _k3,
        grid=(GN,),
        in_specs=[
            pl.BlockSpec((DIM, NT), lambda i: (0, i)),
            pl.BlockSpec((DIM, NT), lambda i: (1, i)),
        ],
        out_specs=[
            pl.BlockSpec((DIM, DIM), lambda i: (0, 0)),
            pl.BlockSpec((DIM, 128), lambda i: (0, 0)),
        ],
        out_shape=[
            jax.ShapeDtypeStruct((DIM, DIM), f32),
            jax.ShapeDtypeStruct((DIM, 128), f32),
        ],
        compiler_params=_ARB,
    )(qkv2d, qkv2d)

    abd = pl.pallas_call(
        _k4,
        out_shape=jax.ShapeDtypeStruct((DIM, DIM), f32),
    )(gram, stats, stats.T, tvec, gsum)

    x1, h0 = pl.pallas_call(
        _k5,
        grid=(GN,),
        in_specs=[
            pl.BlockSpec((DIM, NT), lambda i: (0, i)),
            pl.BlockSpec((DIM, NT), lambda i: (2, i)),
            pl.BlockSpec((DIM, DIM), lambda i: (0, 0)),
            pl.BlockSpec((DIM, DIM), lambda i: (0, 0)),
            pl.BlockSpec((DIM, 1), lambda i: (0, 0)),
            pl.BlockSpec((DIM, 1), lambda i: (0, 0)),
            pl.BlockSpec((2 * HIDDEN, DIM), lambda i: (0, 0)),
        ],
        out_specs=[
            pl.BlockSpec((DIM, NT), lambda i: (0, i)),
            pl.BlockSpec((2 * HIDDEN, NT), lambda i: (0, i)),
        ],
        out_shape=[
            jax.ShapeDtypeStruct((DIM, N), jnp.bfloat16),
            jax.ShapeDtypeStruct((2 * HIDDEN, N), jnp.bfloat16),
        ],
        compiler_params=_PAR,
    )(x2d, qkv2d, abd, pow_, nw, nb, piw)

    m = pl.pallas_call(
        _k6,
        grid=(HIDDEN // CT6,),
        in_specs=[
            pl.BlockSpec((CT6, HW, HW), lambda c: (c, 0, 0)),
            pl.BlockSpec((CT6, HW, HW), lambda c: (c + HIDDEN // CT6, 0, 0)),
            pl.BlockSpec((1, CT6, 9), lambda c: (c, 0, 0),
                         memory_space=pltpu.SMEM),
            pl.BlockSpec((1, CT6, 9), lambda c: (c, 0, 0),
                         memory_space=pltpu.SMEM),
            pl.BlockSpec((1, CT6, 9), lambda c: (c, 0, 0),
                         memory_space=pltpu.SMEM),
            pl.BlockSpec((1, CT6, 9), lambda c: (c, 0, 0),
                         memory_space=pltpu.SMEM),
        ],
        out_specs=pl.BlockSpec((CT6, HW, HW), lambda c: (c, 0, 0)),
        out_shape=jax.ShapeDtypeStruct((HIDDEN, HW, HW), jnp.bfloat16),
        scratch_shapes=[pltpu.VMEM((CT6, HW, HW), f32)] * 7,
        compiler_params=_PAR,
    )(h0.reshape(2 * HIDDEN, HW, HW), h0.reshape(2 * HIDDEN, HW, HW),
      dwa, dwb, dw1, dw2)

    out = pl.pallas_call(
        _k7,
        grid=(GN,),
        in_specs=[
            pl.BlockSpec((DIM, NT), lambda i: (0, i)),
            pl.BlockSpec((HIDDEN, NT), lambda i: (0, i)),
            pl.BlockSpec((DIM, HIDDEN), lambda i: (0, 0)),
        ],
        out_specs=pl.BlockSpec((DIM, NT), lambda i: (0, i)),
        out_shape=jax.ShapeDtypeStruct((DIM, N), f32),
        compiler_params=_PAR,
    )(x1, m.reshape(HIDDEN, N), ipow)

    return out.reshape(1, DIM, HW, HW)


# speedup vs baseline: 1.8917x; 1.1205x over previous
import jax
import jax.numpy as jnp
from jax.experimental import pallas as pl
from jax.experimental.pallas import tpu as pltpu

DIM = 192
HEADS = 8
CH = DIM // HEADS
HIDDEN = int(DIM * 2.66)
HW = 224
N = HW * HW
NT = 1024
GN = N // NT
EPS_LN = 1e-6
EPS_L2 = 1e-12

CT2 = 16
CT6 = 10

_PAR = pltpu.CompilerParams(dimension_semantics=("parallel",))
_ARB = pltpu.CompilerParams(dimension_semantics=("arbitrary",))


def _ln(x, w, b):
    u = jnp.mean(x, axis=0, keepdims=True)
    s = jnp.mean((x - u) * (x - u), axis=0, keepdims=True)
    return w * ((x - u) * jax.lax.rsqrt(s + EPS_LN)) + b


def _k1(x_ref, nw_ref, nb_ref, wqkv_ref, g1w_ref, g1b_ref, g2w_ref, g2b_ref,
        qkv0_ref, gsum_ref):
    xn = _ln(x_ref[...], nw_ref[...], nb_ref[...])
    qkv0_ref[...] = jnp.dot(wqkv_ref[...], xn,
                            preferred_element_type=jnp.float32).astype(jnp.bfloat16)
    gg = jnp.maximum(
        jnp.dot(g1w_ref[...], xn, preferred_element_type=jnp.float32) + g1b_ref[...], 0.0)
    gv = jax.nn.sigmoid(
        jnp.dot(g2w_ref[...], gg, preferred_element_type=jnp.float32) + g2b_ref[...])

    @pl.when(pl.program_id(0) == 0)
    def _():
        gsum_ref[...] = jnp.zeros_like(gsum_ref)

    gsum_ref[...] += jnp.sum(gv)


STRIP = 16
BF16 = jnp.bfloat16


def _shifts_only(src_ref, m1_ref, p1_ref):
    c, hh, ww = src_ref.shape
    z1 = jnp.zeros((c, 1, ww), BF16)
    m1_ref[:, 1:hh, :] = src_ref[:, 0:hh - 1, :].astype(BF16)
    m1_ref[:, 0:1, :] = z1
    p1_ref[:, 0:hh - 1, :] = src_ref[:, 1:hh, :].astype(BF16)
    p1_ref[:, hh - 1:hh, :] = z1


def _dw_strips(cen_ref, m1_ref, p1_ref, ws):
    c, hh, ww = cen_ref.shape
    z = jnp.zeros((STRIP, 1), BF16)
    for s in range(hh // STRIP):
        r0 = s * STRIP
        a0 = m1_ref[:, r0:r0 + STRIP, :]
        a1 = cen_ref[:, r0:r0 + STRIP, :].astype(BF16)
        a2 = p1_ref[:, r0:r0 + STRIP, :]
        for ci in range(c):
            w0, w1, w2 = a0[ci], a1[ci], a2[ci]
            t = ws[ci]
            y0 = w0 * t[0] + w1 * t[3] + w2 * t[6]
            y1 = w0 * t[1] + w1 * t[4] + w2 * t[7]
            y2 = w0 * t[2] + w1 * t[5] + w2 * t[8]
            left = jnp.concatenate([y2[:, 1:], z], axis=1)
            right = jnp.concatenate([z, y0[:, :-1]], axis=1)
            yield ci, r0, y1 + left + right


def _smem_taps(w_ref, c):
    return [[w_ref[0, ci, t].astype(BF16) for t in range(9)] for ci in range(c)]


def _k2(x_ref, w_ref, o_ref, m1_ref, p1_ref):
    ws = _smem_taps(w_ref, x_ref.shape[0])
    _shifts_only(x_ref, m1_ref, p1_ref)
    for ci, r0, acc in _dw_strips(x_ref, m1_ref, p1_ref, ws):
        o_ref[ci, r0:r0 + STRIP, :] = acc.astype(o_ref.dtype)


def _k3(q_ref, k_ref, f_ref, st_ref):
    q = q_ref[...]
    k = k_ref[...]
    q32 = q.astype(jnp.float32)
    k32 = k.astype(jnp.float32)

    @pl.when(pl.program_id(0) == 0)
    def _():
        f_ref[...] = jnp.zeros_like(f_ref)
        st_ref[...] = jnp.zeros_like(st_ref)

    f_ref[...] += jax.lax.dot_general(
        q, k, (((1,), (1,)), ((), ())), preferred_element_type=jnp.float32)
    st_ref[:, 0:1] += jnp.sum(q32 * q32, axis=1, keepdims=True)
    st_ref[:, 1:2] += jnp.sum(k32 * k32, axis=1, keepdims=True)


def _k4(f_ref, st_ref, stt_ref, tv_ref, gs_ref, a_ref):
    dkf = jnp.clip(jnp.floor(CH * gs_ref[0, 0] / N), 1.0, float(CH))

    qn = jnp.maximum(jnp.sqrt(st_ref[:, 0:1]), EPS_L2)
    knt = jnp.maximum(jnp.sqrt(stt_ref[1:2, :]), EPS_L2)
    fn = f_ref[...] / qn / knt * tv_ref[...]

    rows = jax.lax.broadcasted_iota(jnp.int32, (DIM, CH), 0)
    a = jnp.zeros((DIM, CH), jnp.float32)
    for h in range(HEADS):
        in_h = (rows // CH) == h
        a = jnp.where(in_h, fn[:, h * CH:(h + 1) * CH], a)

    iota = jax.lax.broadcasted_iota(jnp.int32, (DIM, CH), 1)
    w = a
    keep = jnp.zeros((DIM, CH), jnp.bool_)
    neg = jnp.float32(-jnp.inf)
    for it in range(CH):
        m = jnp.max(w, axis=1, keepdims=True)
        eq = w == m
        midx = jnp.min(jnp.where(eq, iota, jnp.int32(CH)), axis=1, keepdims=True)
        first = iota == midx
        keep = keep | (first & (jnp.float32(it) < dkf))
        w = jnp.where(first, neg, w)

    s = jnp.where(keep, a, neg)
    mx = jnp.max(s, axis=1, keepdims=True)
    e = jnp.exp(s - mx)
    p = e / jnp.sum(e, axis=1, keepdims=True)

    a_ref[...] = jnp.zeros_like(a_ref)
    for h in range(HEADS):
        a_ref[h * CH:(h + 1) * CH, h * CH:(h + 1) * CH] = p[h * CH:(h + 1) * CH, :]


def _k5(x_ref, v_ref, abd_ref, pow_ref, nw_ref, nb_ref, piw_ref,
        x1_ref, h0_ref):
    av = jnp.dot(abd_ref[...].astype(jnp.bfloat16), v_ref[...],
                 preferred_element_type=jnp.float32)
    x1 = x_ref[...] + jnp.dot(pow_ref[...], av, preferred_element_type=jnp.float32)
    x1_ref[...] = x1.astype(jnp.bfloat16)
    xn1 = _ln(x1, nw_ref[...], nb_ref[...])
    h0_ref[...] = jnp.dot(piw_ref[...], xn1.astype(jnp.bfloat16),
                          preferred_element_type=jnp.float32).astype(jnp.bfloat16)


def _k6(a_ref, b_ref, wa_ref, wb_ref, w1_ref, w2_ref, m_ref,
        m1_ref, p1_ref, sa_ref, sb_ref, n1_ref, q1_ref):
    c = a_ref.shape[0]
    wsa = _smem_taps(wa_ref, c)
    _shifts_only(a_ref, m1_ref, p1_ref)
    for ci, r0, acc in _dw_strips(a_ref, m1_ref, p1_ref, wsa):
        sa_ref[ci, r0:r0 + STRIP, :] = acc
    wsb = _smem_taps(wb_ref, c)
    _shifts_only(b_ref, m1_ref, p1_ref)
    for ci, r0, acc in _dw_strips(b_ref, m1_ref, p1_ref, wsb):
        sb_ref[ci, r0:r0 + STRIP, :] = acc
    ws1 = _smem_taps(w1_ref, c)
    ws2 = _smem_taps(w2_ref, c)
    _shifts_only(sa_ref, m1_ref, p1_ref)
    _shifts_only(sb_ref, n1_ref, q1_ref)
    for (ci, r0, t1), (_, _, t2) in zip(
            _dw_strips(sa_ref, m1_ref, p1_ref, ws1),
            _dw_strips(sb_ref, n1_ref, q1_ref, ws2)):
        v1 = jnp.tanh(t1) + sa_ref[ci, r0:r0 + STRIP, :]
        v2 = jnp.tanh(t2) + sb_ref[ci, r0:r0 + STRIP, :]
        m_ref[ci, r0:r0 + STRIP, :] = (v1 * v2).astype(m_ref.dtype)


def _k7(x1_ref, m_ref, ipow_ref, o_ref):
    o_ref[...] = x1_ref[...].astype(jnp.float32) + jnp.dot(
        ipow_ref[...], m_ref[...], preferred_element_type=jnp.float32)


def kernel(x, norm_w, norm_b, temp, q_w, q_dw, kv_w, kv_dw, po_w, g1_w, g1_b,
           g2_w, g2_b, pi_w, dw_w, dw1_w, dw2_w, ipo_w):
    f32 = jnp.float32
    x2d = x.reshape(DIM, N)
    nw = norm_w.reshape(DIM, 1)
    nb = norm_b.reshape(DIM, 1)
    wqkv = jnp.concatenate([q_w.reshape(DIM, DIM), kv_w.reshape(2 * DIM, DIM)],
                           axis=0)
    g1w = g1_w.reshape(DIM // 2, DIM)
    g1b = g1_b.reshape(DIM // 2, 1)
    g2w = g2_w.reshape(1, DIM // 2)
    g2b = g2_b.reshape(1, 1)
    wdw_qkv = jnp.concatenate([q_dw.reshape(DIM, 9), kv_dw.reshape(2 * DIM, 9)],
                              axis=0).reshape(3 * DIM // CT2, CT2, 9)
    tvec = jnp.repeat(temp.reshape(HEADS), CH).reshape(DIM, 1)
    pow_ = po_w.reshape(DIM, DIM)
    piw = pi_w.reshape(2 * HIDDEN, DIM).astype(jnp.bfloat16)
    dwa = dw_w.reshape(2 * HIDDEN, 9)[:HIDDEN].reshape(HIDDEN // CT6, CT6, 9)
    dwb = dw_w.reshape(2 * HIDDEN, 9)[HIDDEN:].reshape(HIDDEN // CT6, CT6, 9)
    dw1 = dw1_w.reshape(HIDDEN // CT6, CT6, 9)
    dw2 = dw2_w.reshape(HIDDEN // CT6, CT6, 9)
    ipow = ipo_w.reshape(DIM, HIDDEN).astype(jnp.bfloat16)

    qkv0, gsum = pl.pallas_call(
        _k1,
        grid=(GN,),
        in_specs=[
            pl.BlockSpec((DIM, NT), lambda i: (0, i)),
            pl.BlockSpec((DIM, 1), lambda i: (0, 0)),
            pl.BlockSpec((DIM, 1), lambda i: (0, 0)),
            pl.BlockSpec((3 * DIM, DIM), lambda i: (0, 0)),
            pl.BlockSpec((DIM // 2, DIM), lambda i: (0, 0)),
            pl.BlockSpec((DIM // 2, 1), lambda i: (0, 0)),
            pl.BlockSpec((1, DIM // 2), lambda i: (0, 0)),
            pl.BlockSpec((1, 1), lambda i: (0, 0)),
        ],
        out_specs=[
            pl.BlockSpec((3 * DIM, NT), lambda i: (0, i)),
            pl.BlockSpec((8, 128), lambda i: (0, 0)),
        ],
        out_shape=[
            jax.ShapeDtypeStruct((3 * DIM, N), jnp.bfloat16),
            jax.ShapeDtypeStruct((8, 128), f32),
        ],
        compiler_params=_ARB,
    )(x2d, nw, nb, wqkv, g1w, g1b, g2w, g2b)

    qkv = pl.pallas_call(
        _k2,
        grid=(3 * DIM // CT2,),
        in_specs=[
            pl.BlockSpec((CT2, HW, HW), lambda c: (c, 0, 0)),
            pl.BlockSpec((1, CT2, 9), lambda c: (c, 0, 0),
                         memory_space=pltpu.SMEM),
        ],
        out_specs=pl.BlockSpec((CT2, HW, HW), lambda c: (c, 0, 0)),
        out_shape=jax.ShapeDtypeStruct((3 * DIM, HW, HW), jnp.bfloat16),
        scratch_shapes=[pltpu.VMEM((CT2, HW, HW), jnp.bfloat16)] * 2,
        compiler_params=_PAR,
    )(qkv0.reshape(3 * DIM, HW, HW), wdw_qkv)
    qkv2d = qkv.reshape(3 * DIM, N)

    gram, stats = pl.pallas_call(
        _k3,
        grid=(GN,),
        in_specs=[
            pl.BlockSpec((DIM, NT), lambda i: (0, i)),
            pl.BlockSpec((DIM, NT), lambda i: (1, i)),
        ],
        out_specs=[
            pl.BlockSpec((DIM, DIM), lambda i: (0, 0)),
            pl.BlockSpec((DIM, 128), lambda i: (0, 0)),
        ],
        out_shape=[
            jax.ShapeDtypeStruct((DIM, DIM), f32),
            jax.ShapeDtypeStruct((DIM, 128), f32),
        ],
        compiler_params=_ARB,
    )(qkv2d, qkv2d)

    abd = pl.pallas_call(
        _k4,
        out_shape=jax.ShapeDtypeStruct((DIM, DIM), f32),
    )(gram, stats, stats.T, tvec, gsum)

    x1, h0 = pl.pallas_call(
        _k5,
        grid=(GN,),
        in_specs=[
            pl.BlockSpec((DIM, NT), lambda i: (0, i)),
            pl.BlockSpec((DIM, NT), lambda i: (2, i)),
            pl.BlockSpec((DIM, DIM), lambda i: (0, 0)),
            pl.BlockSpec((DIM, DIM), lambda i: (0, 0)),
            pl.BlockSpec((DIM, 1), lambda i: (0, 0)),
            pl.BlockSpec((DIM, 1), lambda i: (0, 0)),
            pl.BlockSpec((2 * HIDDEN, DIM), lambda i: (0, 0)),
        ],
        out_specs=[
            pl.BlockSpec((DIM, NT), lambda i: (0, i)),
            pl.BlockSpec((2 * HIDDEN, NT), lambda i: (0, i)),
        ],
        out_shape=[
            jax.ShapeDtypeStruct((DIM, N), jnp.bfloat16),
            jax.ShapeDtypeStruct((2 * HIDDEN, N), jnp.bfloat16),
        ],
        compiler_params=_PAR,
    )(x2d, qkv2d, abd, pow_, nw, nb, piw)

    m = pl.pallas_call(
        _k6,
        grid=(HIDDEN // CT6,),
        in_specs=[
            pl.BlockSpec((CT6, HW, HW), lambda c: (c, 0, 0)),
            pl.BlockSpec((CT6, HW, HW), lambda c: (c + HIDDEN // CT6, 0, 0)),
            pl.BlockSpec((1, CT6, 9), lambda c: (c, 0, 0),
                         memory_space=pltpu.SMEM),
            pl.BlockSpec((1, CT6, 9), lambda c: (c, 0, 0),
                         memory_space=pltpu.SMEM),
            pl.BlockSpec((1, CT6, 9), lambda c: (c, 0, 0),
                         memory_space=pltpu.SMEM),
            pl.BlockSpec((1, CT6, 9), lambda c: (c, 0, 0),
                         memory_space=pltpu.SMEM),
        ],
        out_specs=pl.BlockSpec((CT6, HW, HW), lambda c: (c, 0, 0)),
        out_shape=jax.ShapeDtypeStruct((HIDDEN, HW, HW), jnp.bfloat16),
        scratch_shapes=[pltpu.VMEM((CT6, HW, HW), jnp.bfloat16)] * 6,
        compiler_params=_PAR,
    )(h0.reshape(2 * HIDDEN, HW, HW), h0.reshape(2 * HIDDEN, HW, HW),
      dwa, dwb, dw1, dw2)

    out = pl.pallas_call(
        _k7,
        grid=(GN,),
        in_specs=[
            pl.BlockSpec((DIM, NT), lambda i: (0, i)),
            pl.BlockSpec((HIDDEN, NT), lambda i: (0, i)),
            pl.BlockSpec((DIM, HIDDEN), lambda i: (0, 0)),
        ],
        out_specs=pl.BlockSpec((DIM, NT), lambda i: (0, i)),
        out_shape=jax.ShapeDtypeStruct((DIM, N), f32),
        compiler_params=_PAR,
    )(x1, m.reshape(HIDDEN, N), ipow)

    return out.reshape(1, DIM, HW, HW)


# speedup vs baseline: 1.9699x; 1.0413x over previous
import jax
import jax.numpy as jnp
from jax.experimental import pallas as pl
from jax.experimental.pallas import tpu as pltpu

DIM = 192
HEADS = 8
CH = DIM // HEADS
HIDDEN = int(DIM * 2.66)
HW = 224
N = HW * HW
NT = 1792
GN = N // NT
EPS_LN = 1e-6
EPS_L2 = 1e-12

CT2 = 16
CT6 = 10

_PAR = pltpu.CompilerParams(dimension_semantics=("parallel",))
_ARB = pltpu.CompilerParams(dimension_semantics=("arbitrary",))


def _ln(x, w, b):
    u = jnp.mean(x, axis=0, keepdims=True)
    s = jnp.mean((x - u) * (x - u), axis=0, keepdims=True)
    return w * ((x - u) * jax.lax.rsqrt(s + EPS_LN)) + b


def _k1(x_ref, nw_ref, nb_ref, wqkv_ref, g1w_ref, g1b_ref, g2w_ref, g2b_ref,
        qkv0_ref, gsum_ref):
    xn = _ln(x_ref[...], nw_ref[...], nb_ref[...])
    qkv0_ref[...] = jnp.dot(wqkv_ref[...], xn,
                            preferred_element_type=jnp.float32).astype(jnp.bfloat16)
    gg = jnp.maximum(
        jnp.dot(g1w_ref[...], xn, preferred_element_type=jnp.float32) + g1b_ref[...], 0.0)
    gv = jax.nn.sigmoid(
        jnp.dot(g2w_ref[...], gg, preferred_element_type=jnp.float32) + g2b_ref[...])

    @pl.when(pl.program_id(0) == 0)
    def _():
        gsum_ref[...] = jnp.zeros_like(gsum_ref)

    gsum_ref[...] += jnp.sum(gv)


STRIP = 16
BF16 = jnp.bfloat16


def _shifts_only(src_ref, m1_ref, p1_ref):
    c, hh, ww = src_ref.shape
    z1 = jnp.zeros((c, 1, ww), BF16)
    m1_ref[:, 1:hh, :] = src_ref[:, 0:hh - 1, :].astype(BF16)
    m1_ref[:, 0:1, :] = z1
    p1_ref[:, 0:hh - 1, :] = src_ref[:, 1:hh, :].astype(BF16)
    p1_ref[:, hh - 1:hh, :] = z1


def _dw_strips(cen_ref, m1_ref, p1_ref, ws):
    c, hh, ww = cen_ref.shape
    z = jnp.zeros((STRIP, 1), BF16)
    for s in range(hh // STRIP):
        r0 = s * STRIP
        a0 = m1_ref[:, r0:r0 + STRIP, :]
        a1 = cen_ref[:, r0:r0 + STRIP, :].astype(BF16)
        a2 = p1_ref[:, r0:r0 + STRIP, :]
        for ci in range(c):
            w0, w1, w2 = a0[ci], a1[ci], a2[ci]
            t = ws[ci]
            y0 = w0 * t[0] + w1 * t[3] + w2 * t[6]
            y1 = w0 * t[1] + w1 * t[4] + w2 * t[7]
            y2 = w0 * t[2] + w1 * t[5] + w2 * t[8]
            left = jnp.concatenate([y2[:, 1:], z], axis=1)
            right = jnp.concatenate([z, y0[:, :-1]], axis=1)
            yield ci, r0, y1 + left + right


def _smem_taps(w_ref, c):
    return [[w_ref[0, ci, t].astype(BF16) for t in range(9)] for ci in range(c)]


def _k2(x_ref, w_ref, o_ref, m1_ref, p1_ref):
    ws = _smem_taps(w_ref, x_ref.shape[0])
    _shifts_only(x_ref, m1_ref, p1_ref)
    for ci, r0, acc in _dw_strips(x_ref, m1_ref, p1_ref, ws):
        o_ref[ci, r0:r0 + STRIP, :] = acc.astype(o_ref.dtype)


def _k3(q_ref, k_ref, f_ref, st_ref):
    q = q_ref[...]
    k = k_ref[...]
    q32 = q.astype(jnp.float32)
    k32 = k.astype(jnp.float32)

    @pl.when(pl.program_id(0) == 0)
    def _():
        f_ref[...] = jnp.zeros_like(f_ref)
        st_ref[...] = jnp.zeros_like(st_ref)

    f_ref[...] += jax.lax.dot_general(
        q, k, (((1,), (1,)), ((), ())), preferred_element_type=jnp.float32)
    st_ref[:, 0:1] += jnp.sum(q32 * q32, axis=1, keepdims=True)
    st_ref[:, 1:2] += jnp.sum(k32 * k32, axis=1, keepdims=True)


def _k4(f_ref, st_ref, stt_ref, tv_ref, gs_ref, a_ref):
    dkf = jnp.clip(jnp.floor(CH * gs_ref[0, 0] / N), 1.0, float(CH))

    qn = jnp.maximum(jnp.sqrt(st_ref[:, 0:1]), EPS_L2)
    knt = jnp.maximum(jnp.sqrt(stt_ref[1:2, :]), EPS_L2)
    fn = f_ref[...] / qn / knt * tv_ref[...]

    rows = jax.lax.broadcasted_iota(jnp.int32, (DIM, CH), 0)
    a = jnp.zeros((DIM, CH), jnp.float32)
    for h in range(HEADS):
        in_h = (rows // CH) == h
        a = jnp.where(in_h, fn[:, h * CH:(h + 1) * CH], a)

    iota = jax.lax.broadcasted_iota(jnp.int32, (DIM, CH), 1)
    w = a
    keep = jnp.zeros((DIM, CH), jnp.bool_)
    neg = jnp.float32(-jnp.inf)
    for it in range(CH):
        m = jnp.max(w, axis=1, keepdims=True)
        eq = w == m
        midx = jnp.min(jnp.where(eq, iota, jnp.int32(CH)), axis=1, keepdims=True)
        first = iota == midx
        keep = keep | (first & (jnp.float32(it) < dkf))
        w = jnp.where(first, neg, w)

    s = jnp.where(keep, a, neg)
    mx = jnp.max(s, axis=1, keepdims=True)
    e = jnp.exp(s - mx)
    p = e / jnp.sum(e, axis=1, keepdims=True)

    a_ref[...] = jnp.zeros_like(a_ref)
    for h in range(HEADS):
        a_ref[h * CH:(h + 1) * CH, h * CH:(h + 1) * CH] = p[h * CH:(h + 1) * CH, :]


def _k5(x_ref, v_ref, abd_ref, pow_ref, nw_ref, nb_ref, piw_ref,
        x1_ref, h0_ref):
    av = jnp.dot(abd_ref[...].astype(jnp.bfloat16), v_ref[...],
                 preferred_element_type=jnp.float32)
    x1 = x_ref[...] + jnp.dot(pow_ref[...], av, preferred_element_type=jnp.float32)
    x1_ref[...] = x1.astype(jnp.bfloat16)
    xn1 = _ln(x1, nw_ref[...], nb_ref[...])
    h0_ref[...] = jnp.dot(piw_ref[...], xn1.astype(jnp.bfloat16),
                          preferred_element_type=jnp.float32).astype(jnp.bfloat16)


def _k6(a_ref, b_ref, wa_ref, wb_ref, w1_ref, w2_ref, m_ref,
        m1_ref, p1_ref, sa_ref, sb_ref, n1_ref, q1_ref):
    c = a_ref.shape[0]
    wsa = _smem_taps(wa_ref, c)
    _shifts_only(a_ref, m1_ref, p1_ref)
    for ci, r0, acc in _dw_strips(a_ref, m1_ref, p1_ref, wsa):
        sa_ref[ci, r0:r0 + STRIP, :] = acc
    wsb = _smem_taps(wb_ref, c)
    _shifts_only(b_ref, m1_ref, p1_ref)
    for ci, r0, acc in _dw_strips(b_ref, m1_ref, p1_ref, wsb):
        sb_ref[ci, r0:r0 + STRIP, :] = acc
    ws1 = _smem_taps(w1_ref, c)
    ws2 = _smem_taps(w2_ref, c)
    _shifts_only(sa_ref, m1_ref, p1_ref)
    _shifts_only(sb_ref, n1_ref, q1_ref)
    for (ci, r0, t1), (_, _, t2) in zip(
            _dw_strips(sa_ref, m1_ref, p1_ref, ws1),
            _dw_strips(sb_ref, n1_ref, q1_ref, ws2)):
        v1 = jnp.tanh(t1) + sa_ref[ci, r0:r0 + STRIP, :]
        v2 = jnp.tanh(t2) + sb_ref[ci, r0:r0 + STRIP, :]
        m_ref[ci, r0:r0 + STRIP, :] = (v1 * v2).astype(m_ref.dtype)


def _k7(x1_ref, m_ref, ipow_ref, o_ref):
    o_ref[...] = x1_ref[...].astype(jnp.float32) + jnp.dot(
        ipow_ref[...], m_ref[...], preferred_element_type=jnp.float32)


def kernel(x, norm_w, norm_b, temp, q_w, q_dw, kv_w, kv_dw, po_w, g1_w, g1_b,
           g2_w, g2_b, pi_w, dw_w, dw1_w, dw2_w, ipo_w):
    f32 = jnp.float32
    x2d = x.reshape(DIM, N)
    nw = norm_w.reshape(DIM, 1)
    nb = norm_b.reshape(DIM, 1)
    wqkv = jnp.concatenate([q_w.reshape(DIM, DIM), kv_w.reshape(2 * DIM, DIM)],
                           axis=0)
    g1w = g1_w.reshape(DIM // 2, DIM)
    g1b = g1_b.reshape(DIM // 2, 1)
    g2w = g2_w.reshape(1, DIM // 2)
    g2b = g2_b.reshape(1, 1)
    wdw_qkv = jnp.concatenate([q_dw.reshape(DIM, 9), kv_dw.reshape(2 * DIM, 9)],
                              axis=0).reshape(3 * DIM // CT2, CT2, 9)
    tvec = jnp.repeat(temp.reshape(HEADS), CH).reshape(DIM, 1)
    pow_ = po_w.reshape(DIM, DIM)
    piw = pi_w.reshape(2 * HIDDEN, DIM).astype(jnp.bfloat16)
    dwa = dw_w.reshape(2 * HIDDEN, 9)[:HIDDEN].reshape(HIDDEN // CT6, CT6, 9)
    dwb = dw_w.reshape(2 * HIDDEN, 9)[HIDDEN:].reshape(HIDDEN // CT6, CT6, 9)
    dw1 = dw1_w.reshape(HIDDEN // CT6, CT6, 9)
    dw2 = dw2_w.reshape(HIDDEN // CT6, CT6, 9)
    ipow = ipo_w.reshape(DIM, HIDDEN).astype(jnp.bfloat16)

    qkv0, gsum = pl.pallas_call(
        _k1,
        grid=(GN,),
        in_specs=[
            pl.BlockSpec((DIM, NT), lambda i: (0, i)),
            pl.BlockSpec((DIM, 1), lambda i: (0, 0)),
            pl.BlockSpec((DIM, 1), lambda i: (0, 0)),
            pl.BlockSpec((3 * DIM, DIM), lambda i: (0, 0)),
            pl.BlockSpec((DIM // 2, DIM), lambda i: (0, 0)),
            pl.BlockSpec((DIM // 2, 1), lambda i: (0, 0)),
            pl.BlockSpec((1, DIM // 2), lambda i: (0, 0)),
            pl.BlockSpec((1, 1), lambda i: (0, 0)),
        ],
        out_specs=[
            pl.BlockSpec((3 * DIM, NT), lambda i: (0, i)),
            pl.BlockSpec((8, 128), lambda i: (0, 0)),
        ],
        out_shape=[
            jax.ShapeDtypeStruct((3 * DIM, N), jnp.bfloat16),
            jax.ShapeDtypeStruct((8, 128), f32),
        ],
        compiler_params=_ARB,
    )(x2d, nw, nb, wqkv, g1w, g1b, g2w, g2b)

    qkv = pl.pallas_call(
        _k2,
        grid=(3 * DIM // CT2,),
        in_specs=[
            pl.BlockSpec((CT2, HW, HW), lambda c: (c, 0, 0)),
            pl.BlockSpec((1, CT2, 9), lambda c: (c, 0, 0),
                         memory_space=pltpu.SMEM),
        ],
        out_specs=pl.BlockSpec((CT2, HW, HW), lambda c: (c, 0, 0)),
        out_shape=jax.ShapeDtypeStruct((3 * DIM, HW, HW), jnp.bfloat16),
        scratch_shapes=[pltpu.VMEM((CT2, HW, HW), jnp.bfloat16)] * 2,
        compiler_params=_PAR,
    )(qkv0.reshape(3 * DIM, HW, HW), wdw_qkv)
    qkv2d = qkv.reshape(3 * DIM, N)

    gram, stats = pl.pallas_call(
        _k3,
        grid=(GN,),
        in_specs=[
            pl.BlockSpec((DIM, NT), lambda i: (0, i)),
            pl.BlockSpec((DIM, NT), lambda i: (1, i)),
        ],
        out_specs=[
            pl.BlockSpec((DIM, DIM), lambda i: (0, 0)),
            pl.BlockSpec((DIM, 128), lambda i: (0, 0)),
        ],
        out_shape=[
            jax.ShapeDtypeStruct((DIM, DIM), f32),
            jax.ShapeDtypeStruct((DIM, 128), f32),
        ],
        compiler_params=_ARB,
    )(qkv2d, qkv2d)

    abd = pl.pallas_call(
        _k4,
        out_shape=jax.ShapeDtypeStruct((DIM, DIM), f32),
    )(gram, stats, stats.T, tvec, gsum)

    x1, h0 = pl.pallas_call(
        _k5,
        grid=(GN,),
        in_specs=[
            pl.BlockSpec((DIM, NT), lambda i: (0, i)),
            pl.BlockSpec((DIM, NT), lambda i: (2, i)),
            pl.BlockSpec((DIM, DIM), lambda i: (0, 0)),
            pl.BlockSpec((DIM, DIM), lambda i: (0, 0)),
            pl.BlockSpec((DIM, 1), lambda i: (0, 0)),
            pl.BlockSpec((DIM, 1), lambda i: (0, 0)),
            pl.BlockSpec((2 * HIDDEN, DIM), lambda i: (0, 0)),
        ],
        out_specs=[
            pl.BlockSpec((DIM, NT), lambda i: (0, i)),
            pl.BlockSpec((2 * HIDDEN, NT), lambda i: (0, i)),
        ],
        out_shape=[
            jax.ShapeDtypeStruct((DIM, N), jnp.bfloat16),
            jax.ShapeDtypeStruct((2 * HIDDEN, N), jnp.bfloat16),
        ],
        compiler_params=_PAR,
    )(x2d, qkv2d, abd, pow_, nw, nb, piw)

    m = pl.pallas_call(
        _k6,
        grid=(HIDDEN // CT6,),
        in_specs=[
            pl.BlockSpec((CT6, HW, HW), lambda c: (c, 0, 0)),
            pl.BlockSpec((CT6, HW, HW), lambda c: (c + HIDDEN // CT6, 0, 0)),
            pl.BlockSpec((1, CT6, 9), lambda c: (c, 0, 0),
                         memory_space=pltpu.SMEM),
            pl.BlockSpec((1, CT6, 9), lambda c: (c, 0, 0),
                         memory_space=pltpu.SMEM),
            pl.BlockSpec((1, CT6, 9), lambda c: (c, 0, 0),
                         memory_space=pltpu.SMEM),
            pl.BlockSpec((1, CT6, 9), lambda c: (c, 0, 0),
                         memory_space=pltpu.SMEM),
        ],
        out_specs=pl.BlockSpec((CT6, HW, HW), lambda c: (c, 0, 0)),
        out_shape=jax.ShapeDtypeStruct((HIDDEN, HW, HW), jnp.bfloat16),
        scratch_shapes=[pltpu.VMEM((CT6, HW, HW), jnp.bfloat16)] * 6,
        compiler_params=_PAR,
    )(h0.reshape(2 * HIDDEN, HW, HW), h0.reshape(2 * HIDDEN, HW, HW),
      dwa, dwb, dw1, dw2)

    out = pl.pallas_call(
        _k7,
        grid=(GN,),
        in_specs=[
            pl.BlockSpec((DIM, NT), lambda i: (0, i)),
            pl.BlockSpec((HIDDEN, NT), lambda i: (0, i)),
            pl.BlockSpec((DIM, HIDDEN), lambda i: (0, 0)),
        ],
        out_specs=pl.BlockSpec((DIM, NT), lambda i: (0, i)),
        out_shape=jax.ShapeDtypeStruct((DIM, N), f32),
        compiler_params=_PAR,
    )(x1, m.reshape(HIDDEN, N), ipow)

    return out.reshape(1, DIM, HW, HW)


# speedup vs baseline: 2.0157x; 1.0232x over previous
import jax
import jax.numpy as jnp
from jax.experimental import pallas as pl
from jax.experimental.pallas import tpu as pltpu

DIM = 192
HEADS = 8
CH = DIM // HEADS
HIDDEN = int(DIM * 2.66)
HW = 224
N = HW * HW
NT = 3584
GN = N // NT
EPS_LN = 1e-6
EPS_L2 = 1e-12

CT2 = 16
CT6 = 10

_PAR = pltpu.CompilerParams(dimension_semantics=("parallel",))
_ARB = pltpu.CompilerParams(dimension_semantics=("arbitrary",))


def _ln(x, w, b):
    u = jnp.mean(x, axis=0, keepdims=True)
    s = jnp.mean((x - u) * (x - u), axis=0, keepdims=True)
    return w * ((x - u) * jax.lax.rsqrt(s + EPS_LN)) + b


def _k1(x_ref, nw_ref, nb_ref, wqkv_ref, g1w_ref, g1b_ref, g2w_ref, g2b_ref,
        qkv0_ref, gsum_ref):
    xn = _ln(x_ref[...], nw_ref[...], nb_ref[...])
    qkv0_ref[...] = jnp.dot(wqkv_ref[...], xn,
                            preferred_element_type=jnp.float32).astype(jnp.bfloat16)
    gg = jnp.maximum(
        jnp.dot(g1w_ref[...], xn, preferred_element_type=jnp.float32) + g1b_ref[...], 0.0)
    gv = jax.nn.sigmoid(
        jnp.dot(g2w_ref[...], gg, preferred_element_type=jnp.float32) + g2b_ref[...])

    @pl.when(pl.program_id(0) == 0)
    def _():
        gsum_ref[...] = jnp.zeros_like(gsum_ref)

    gsum_ref[...] += jnp.sum(gv)


STRIP = 16
BF16 = jnp.bfloat16


def _shifts_only(src_ref, m1_ref, p1_ref):
    c, hh, ww = src_ref.shape
    z1 = jnp.zeros((c, 1, ww), BF16)
    m1_ref[:, 1:hh, :] = src_ref[:, 0:hh - 1, :].astype(BF16)
    m1_ref[:, 0:1, :] = z1
    p1_ref[:, 0:hh - 1, :] = src_ref[:, 1:hh, :].astype(BF16)
    p1_ref[:, hh - 1:hh, :] = z1


def _dw_strips(cen_ref, m1_ref, p1_ref, ws):
    c, hh, ww = cen_ref.shape
    z = jnp.zeros((STRIP, 1), BF16)
    for s in range(hh // STRIP):
        r0 = s * STRIP
        a0 = m1_ref[:, r0:r0 + STRIP, :]
        a1 = cen_ref[:, r0:r0 + STRIP, :].astype(BF16)
        a2 = p1_ref[:, r0:r0 + STRIP, :]
        for ci in range(c):
            w0, w1, w2 = a0[ci], a1[ci], a2[ci]
            t = ws[ci]
            y0 = w0 * t[0] + w1 * t[3] + w2 * t[6]
            y1 = w0 * t[1] + w1 * t[4] + w2 * t[7]
            y2 = w0 * t[2] + w1 * t[5] + w2 * t[8]
            left = jnp.concatenate([y2[:, 1:], z], axis=1)
            right = jnp.concatenate([z, y0[:, :-1]], axis=1)
            yield ci, r0, y1 + left + right


def _smem_taps(w_ref, c):
    return [[w_ref[0, ci, t].astype(BF16) for t in range(9)] for ci in range(c)]


def _k2(x_ref, w_ref, o_ref, m1_ref, p1_ref):
    ws = _smem_taps(w_ref, x_ref.shape[0])
    _shifts_only(x_ref, m1_ref, p1_ref)
    for ci, r0, acc in _dw_strips(x_ref, m1_ref, p1_ref, ws):
        o_ref[ci, r0:r0 + STRIP, :] = acc.astype(o_ref.dtype)


def _k3(q_ref, k_ref, f_ref, st_ref):
    q = q_ref[...]
    k = k_ref[...]
    q32 = q.astype(jnp.float32)
    k32 = k.astype(jnp.float32)

    @pl.when(pl.program_id(0) == 0)
    def _():
        f_ref[...] = jnp.zeros_like(f_ref)
        st_ref[...] = jnp.zeros_like(st_ref)

    f_ref[...] += jax.lax.dot_general(
        q, k, (((1,), (1,)), ((), ())), preferred_element_type=jnp.float32)
    st_ref[:, 0:1] += jnp.sum(q32 * q32, axis=1, keepdims=True)
    st_ref[:, 1:2] += jnp.sum(k32 * k32, axis=1, keepdims=True)


def _k4(f_ref, st_ref, stt_ref, tv_ref, gs_ref, a_ref):
    dkf = jnp.clip(jnp.floor(CH * gs_ref[0, 0] / N), 1.0, float(CH))

    qn = jnp.maximum(jnp.sqrt(st_ref[:, 0:1]), EPS_L2)
    knt = jnp.maximum(jnp.sqrt(stt_ref[1:2, :]), EPS_L2)
    fn = f_ref[...] / qn / knt * tv_ref[...]

    rows = jax.lax.broadcasted_iota(jnp.int32, (DIM, CH), 0)
    a = jnp.zeros((DIM, CH), jnp.float32)
    for h in range(HEADS):
        in_h = (rows // CH) == h
        a = jnp.where(in_h, fn[:, h * CH:(h + 1) * CH], a)

    iota = jax.lax.broadcasted_iota(jnp.int32, (DIM, CH), 1)
    w = a
    keep = jnp.zeros((DIM, CH), jnp.bool_)
    neg = jnp.float32(-jnp.inf)
    for it in range(CH):
        m = jnp.max(w, axis=1, keepdims=True)
        eq = w == m
        midx = jnp.min(jnp.where(eq, iota, jnp.int32(CH)), axis=1, keepdims=True)
        first = iota == midx
        keep = keep | (first & (jnp.float32(it) < dkf))
        w = jnp.where(first, neg, w)

    s = jnp.where(keep, a, neg)
    mx = jnp.max(s, axis=1, keepdims=True)
    e = jnp.exp(s - mx)
    p = e / jnp.sum(e, axis=1, keepdims=True)

    a_ref[...] = jnp.zeros_like(a_ref)
    for h in range(HEADS):
        a_ref[h * CH:(h + 1) * CH, h * CH:(h + 1) * CH] = p[h * CH:(h + 1) * CH, :]


def _k5(x_ref, v_ref, abd_ref, pow_ref, nw_ref, nb_ref, piw_ref,
        x1_ref, h0_ref):
    av = jnp.dot(abd_ref[...].astype(jnp.bfloat16), v_ref[...],
                 preferred_element_type=jnp.float32)
    x1 = x_ref[...] + jnp.dot(pow_ref[...], av, preferred_element_type=jnp.float32)
    x1_ref[...] = x1.astype(jnp.bfloat16)
    xn1 = _ln(x1, nw_ref[...], nb_ref[...])
    h0_ref[...] = jnp.dot(piw_ref[...], xn1.astype(jnp.bfloat16),
                          preferred_element_type=jnp.float32).astype(jnp.bfloat16)


def _k6(a_ref, b_ref, wa_ref, wb_ref, w1_ref, w2_ref, m_ref,
        m1_ref, p1_ref, sa_ref, sb_ref, n1_ref, q1_ref):
    c = a_ref.shape[0]
    wsa = _smem_taps(wa_ref, c)
    _shifts_only(a_ref, m1_ref, p1_ref)
    for ci, r0, acc in _dw_strips(a_ref, m1_ref, p1_ref, wsa):
        sa_ref[ci, r0:r0 + STRIP, :] = acc
    wsb = _smem_taps(wb_ref, c)
    _shifts_only(b_ref, m1_ref, p1_ref)
    for ci, r0, acc in _dw_strips(b_ref, m1_ref, p1_ref, wsb):
        sb_ref[ci, r0:r0 + STRIP, :] = acc
    ws1 = _smem_taps(w1_ref, c)
    ws2 = _smem_taps(w2_ref, c)
    _shifts_only(sa_ref, m1_ref, p1_ref)
    _shifts_only(sb_ref, n1_ref, q1_ref)
    for (ci, r0, t1), (_, _, t2) in zip(
            _dw_strips(sa_ref, m1_ref, p1_ref, ws1),
            _dw_strips(sb_ref, n1_ref, q1_ref, ws2)):
        v1 = jnp.tanh(t1) + sa_ref[ci, r0:r0 + STRIP, :]
        v2 = jnp.tanh(t2) + sb_ref[ci, r0:r0 + STRIP, :]
        m_ref[ci, r0:r0 + STRIP, :] = (v1 * v2).astype(m_ref.dtype)


def _k7(x1_ref, m_ref, ipow_ref, o_ref):
    o_ref[...] = x1_ref[...].astype(jnp.float32) + jnp.dot(
        ipow_ref[...], m_ref[...], preferred_element_type=jnp.float32)


def kernel(x, norm_w, norm_b, temp, q_w, q_dw, kv_w, kv_dw, po_w, g1_w, g1_b,
           g2_w, g2_b, pi_w, dw_w, dw1_w, dw2_w, ipo_w):
    f32 = jnp.float32
    x2d = x.reshape(DIM, N)
    nw = norm_w.reshape(DIM, 1)
    nb = norm_b.reshape(DIM, 1)
    wqkv = jnp.concatenate([q_w.reshape(DIM, DIM), kv_w.reshape(2 * DIM, DIM)],
                           axis=0)
    g1w = g1_w.reshape(DIM // 2, DIM)
    g1b = g1_b.reshape(DIM // 2, 1)
    g2w = g2_w.reshape(1, DIM // 2)
    g2b = g2_b.reshape(1, 1)
    wdw_qkv = jnp.concatenate([q_dw.reshape(DIM, 9), kv_dw.reshape(2 * DIM, 9)],
                              axis=0).reshape(3 * DIM // CT2, CT2, 9)
    tvec = jnp.repeat(temp.reshape(HEADS), CH).reshape(DIM, 1)
    pow_ = po_w.reshape(DIM, DIM)
    piw = pi_w.reshape(2 * HIDDEN, DIM).astype(jnp.bfloat16)
    dwa = dw_w.reshape(2 * HIDDEN, 9)[:HIDDEN].reshape(HIDDEN // CT6, CT6, 9)
    dwb = dw_w.reshape(2 * HIDDEN, 9)[HIDDEN:].reshape(HIDDEN // CT6, CT6, 9)
    dw1 = dw1_w.reshape(HIDDEN // CT6, CT6, 9)
    dw2 = dw2_w.reshape(HIDDEN // CT6, CT6, 9)
    ipow = ipo_w.reshape(DIM, HIDDEN).astype(jnp.bfloat16)

    qkv0, gsum = pl.pallas_call(
        _k1,
        grid=(GN,),
        in_specs=[
            pl.BlockSpec((DIM, NT), lambda i: (0, i)),
            pl.BlockSpec((DIM, 1), lambda i: (0, 0)),
            pl.BlockSpec((DIM, 1), lambda i: (0, 0)),
            pl.BlockSpec((3 * DIM, DIM), lambda i: (0, 0)),
            pl.BlockSpec((DIM // 2, DIM), lambda i: (0, 0)),
            pl.BlockSpec((DIM // 2, 1), lambda i: (0, 0)),
            pl.BlockSpec((1, DIM // 2), lambda i: (0, 0)),
            pl.BlockSpec((1, 1), lambda i: (0, 0)),
        ],
        out_specs=[
            pl.BlockSpec((3 * DIM, NT), lambda i: (0, i)),
            pl.BlockSpec((8, 128), lambda i: (0, 0)),
        ],
        out_shape=[
            jax.ShapeDtypeStruct((3 * DIM, N), jnp.bfloat16),
            jax.ShapeDtypeStruct((8, 128), f32),
        ],
        compiler_params=_ARB,
    )(x2d, nw, nb, wqkv, g1w, g1b, g2w, g2b)

    qkv = pl.pallas_call(
        _k2,
        grid=(3 * DIM // CT2,),
        in_specs=[
            pl.BlockSpec((CT2, HW, HW), lambda c: (c, 0, 0)),
            pl.BlockSpec((1, CT2, 9), lambda c: (c, 0, 0),
                         memory_space=pltpu.SMEM),
        ],
        out_specs=pl.BlockSpec((CT2, HW, HW), lambda c: (c, 0, 0)),
        out_shape=jax.ShapeDtypeStruct((3 * DIM, HW, HW), jnp.bfloat16),
        scratch_shapes=[pltpu.VMEM((CT2, HW, HW), jnp.bfloat16)] * 2,
        compiler_params=_PAR,
    )(qkv0.reshape(3 * DIM, HW, HW), wdw_qkv)
    qkv2d = qkv.reshape(3 * DIM, N)

    gram, stats = pl.pallas_call(
        _k3,
        grid=(GN,),
        in_specs=[
            pl.BlockSpec((DIM, NT), lambda i: (0, i)),
            pl.BlockSpec((DIM, NT), lambda i: (1, i)),
        ],
        out_specs=[
            pl.BlockSpec((DIM, DIM), lambda i: (0, 0)),
            pl.BlockSpec((DIM, 128), lambda i: (0, 0)),
        ],
        out_shape=[
            jax.ShapeDtypeStruct((DIM, DIM), f32),
            jax.ShapeDtypeStruct((DIM, 128), f32),
        ],
        compiler_params=_ARB,
    )(qkv2d, qkv2d)

    abd = pl.pallas_call(
        _k4,
        out_shape=jax.ShapeDtypeStruct((DIM, DIM), f32),
    )(gram, stats, stats.T, tvec, gsum)

    x1, h0 = pl.pallas_call(
        _k5,
        grid=(GN,),
        in_specs=[
            pl.BlockSpec((DIM, NT), lambda i: (0, i)),
            pl.BlockSpec((DIM, NT), lambda i: (2, i)),
            pl.BlockSpec((DIM, DIM), lambda i: (0, 0)),
            pl.BlockSpec((DIM, DIM), lambda i: (0, 0)),
            pl.BlockSpec((DIM, 1), lambda i: (0, 0)),
            pl.BlockSpec((DIM, 1), lambda i: (0, 0)),
            pl.BlockSpec((2 * HIDDEN, DIM), lambda i: (0, 0)),
        ],
        out_specs=[
            pl.BlockSpec((DIM, NT), lambda i: (0, i)),
            pl.BlockSpec((2 * HIDDEN, NT), lambda i: (0, i)),
        ],
        out_shape=[
            jax.ShapeDtypeStruct((DIM, N), jnp.bfloat16),
            jax.ShapeDtypeStruct((2 * HIDDEN, N), jnp.bfloat16),
        ],
        compiler_params=_PAR,
    )(x2d, qkv2d, abd, pow_, nw, nb, piw)

    m = pl.pallas_call(
        _k6,
        grid=(HIDDEN // CT6,),
        in_specs=[
            pl.BlockSpec((CT6, HW, HW), lambda c: (c, 0, 0)),
            pl.BlockSpec((CT6, HW, HW), lambda c: (c + HIDDEN // CT6, 0, 0)),
            pl.BlockSpec((1, CT6, 9), lambda c: (c, 0, 0),
                         memory_space=pltpu.SMEM),
            pl.BlockSpec((1, CT6, 9), lambda c: (c, 0, 0),
                         memory_space=pltpu.SMEM),
            pl.BlockSpec((1, CT6, 9), lambda c: (c, 0, 0),
                         memory_space=pltpu.SMEM),
            pl.BlockSpec((1, CT6, 9), lambda c: (c, 0, 0),
                         memory_space=pltpu.SMEM),
        ],
        out_specs=pl.BlockSpec((CT6, HW, HW), lambda c: (c, 0, 0)),
        out_shape=jax.ShapeDtypeStruct((HIDDEN, HW, HW), jnp.bfloat16),
        scratch_shapes=[pltpu.VMEM((CT6, HW, HW), jnp.bfloat16)] * 6,
        compiler_params=_PAR,
    )(h0.reshape(2 * HIDDEN, HW, HW), h0.reshape(2 * HIDDEN, HW, HW),
      dwa, dwb, dw1, dw2)

    out = pl.pallas_call(
        _k7,
        grid=(GN,),
        in_specs=[
            pl.BlockSpec((DIM, NT), lambda i: (0, i)),
            pl.BlockSpec((HIDDEN, NT), lambda i: (0, i)),
            pl.BlockSpec((DIM, HIDDEN), lambda i: (0, 0)),
        ],
        out_specs=pl.BlockSpec((DIM, NT), lambda i: (0, i)),
        out_shape=jax.ShapeDtypeStruct((DIM, N), f32),
        compiler_params=_PAR,
    )(x1, m.reshape(HIDDEN, N), ipow)

    return out.reshape(1, DIM, HW, HW)


# speedup vs baseline: 2.0271x; 1.0057x over previous
import jax
import jax.numpy as jnp
from jax.experimental import pallas as pl
from jax.experimental.pallas import tpu as pltpu

DIM = 192
HEADS = 8
CH = DIM // HEADS
HIDDEN = int(DIM * 2.66)
HW = 224
N = HW * HW
NT = 7168
GN = N // NT
EPS_LN = 1e-6
EPS_L2 = 1e-12

CT2 = 16
CT6 = 10

_PAR = pltpu.CompilerParams(dimension_semantics=("parallel",))
_ARB = pltpu.CompilerParams(dimension_semantics=("arbitrary",))


def _ln(x, w, b):
    u = jnp.mean(x, axis=0, keepdims=True)
    s = jnp.mean((x - u) * (x - u), axis=0, keepdims=True)
    return w * ((x - u) * jax.lax.rsqrt(s + EPS_LN)) + b


def _k1(x_ref, nw_ref, nb_ref, wqkv_ref, g1w_ref, g1b_ref, g2w_ref, g2b_ref,
        qkv0_ref, gsum_ref):
    xn = _ln(x_ref[...], nw_ref[...], nb_ref[...])
    qkv0_ref[...] = jnp.dot(wqkv_ref[...], xn,
                            preferred_element_type=jnp.float32).astype(jnp.bfloat16)
    gg = jnp.maximum(
        jnp.dot(g1w_ref[...], xn, preferred_element_type=jnp.float32) + g1b_ref[...], 0.0)
    gv = jax.nn.sigmoid(
        jnp.dot(g2w_ref[...], gg, preferred_element_type=jnp.float32) + g2b_ref[...])

    @pl.when(pl.program_id(0) == 0)
    def _():
        gsum_ref[...] = jnp.zeros_like(gsum_ref)

    gsum_ref[...] += jnp.sum(gv)


STRIP = 16
BF16 = jnp.bfloat16


def _shifts_only(src_ref, m1_ref, p1_ref):
    c, hh, ww = src_ref.shape
    z1 = jnp.zeros((c, 1, ww), BF16)
    m1_ref[:, 1:hh, :] = src_ref[:, 0:hh - 1, :].astype(BF16)
    m1_ref[:, 0:1, :] = z1
    p1_ref[:, 0:hh - 1, :] = src_ref[:, 1:hh, :].astype(BF16)
    p1_ref[:, hh - 1:hh, :] = z1


def _dw_strips(cen_ref, m1_ref, p1_ref, ws):
    c, hh, ww = cen_ref.shape
    z = jnp.zeros((STRIP, 1), BF16)
    for s in range(hh // STRIP):
        r0 = s * STRIP
        a0 = m1_ref[:, r0:r0 + STRIP, :]
        a1 = cen_ref[:, r0:r0 + STRIP, :].astype(BF16)
        a2 = p1_ref[:, r0:r0 + STRIP, :]
        for ci in range(c):
            w0, w1, w2 = a0[ci], a1[ci], a2[ci]
            t = ws[ci]
            y0 = w0 * t[0] + w1 * t[3] + w2 * t[6]
            y1 = w0 * t[1] + w1 * t[4] + w2 * t[7]
            y2 = w0 * t[2] + w1 * t[5] + w2 * t[8]
            left = jnp.concatenate([y2[:, 1:], z], axis=1)
            right = jnp.concatenate([z, y0[:, :-1]], axis=1)
            yield ci, r0, y1 + left + right


def _smem_taps(w_ref, c):
    return [[w_ref[0, ci, t].astype(BF16) for t in range(9)] for ci in range(c)]


def _k2(x_ref, w_ref, o_ref, m1_ref, p1_ref):
    ws = _smem_taps(w_ref, x_ref.shape[0])
    _shifts_only(x_ref, m1_ref, p1_ref)
    for ci, r0, acc in _dw_strips(x_ref, m1_ref, p1_ref, ws):
        o_ref[ci, r0:r0 + STRIP, :] = acc.astype(o_ref.dtype)


def _k3(q_ref, k_ref, f_ref, st_ref):
    q = q_ref[...]
    k = k_ref[...]
    q32 = q.astype(jnp.float32)
    k32 = k.astype(jnp.float32)

    @pl.when(pl.program_id(0) == 0)
    def _():
        f_ref[...] = jnp.zeros_like(f_ref)
        st_ref[...] = jnp.zeros_like(st_ref)

    f_ref[...] += jax.lax.dot_general(
        q, k, (((1,), (1,)), ((), ())), preferred_element_type=jnp.float32)
    st_ref[:, 0:1] += jnp.sum(q32 * q32, axis=1, keepdims=True)
    st_ref[:, 1:2] += jnp.sum(k32 * k32, axis=1, keepdims=True)


def _k4(f_ref, st_ref, stt_ref, tv_ref, gs_ref, a_ref):
    dkf = jnp.clip(jnp.floor(CH * gs_ref[0, 0] / N), 1.0, float(CH))

    qn = jnp.maximum(jnp.sqrt(st_ref[:, 0:1]), EPS_L2)
    knt = jnp.maximum(jnp.sqrt(stt_ref[1:2, :]), EPS_L2)
    fn = f_ref[...] / qn / knt * tv_ref[...]

    rows = jax.lax.broadcasted_iota(jnp.int32, (DIM, CH), 0)
    a = jnp.zeros((DIM, CH), jnp.float32)
    for h in range(HEADS):
        in_h = (rows // CH) == h
        a = jnp.where(in_h, fn[:, h * CH:(h + 1) * CH], a)

    iota = jax.lax.broadcasted_iota(jnp.int32, (DIM, CH), 1)
    w = a
    keep = jnp.zeros((DIM, CH), jnp.bool_)
    neg = jnp.float32(-jnp.inf)
    for it in range(CH):
        m = jnp.max(w, axis=1, keepdims=True)
        eq = w == m
        midx = jnp.min(jnp.where(eq, iota, jnp.int32(CH)), axis=1, keepdims=True)
        first = iota == midx
        keep = keep | (first & (jnp.float32(it) < dkf))
        w = jnp.where(first, neg, w)

    s = jnp.where(keep, a, neg)
    mx = jnp.max(s, axis=1, keepdims=True)
    e = jnp.exp(s - mx)
    p = e / jnp.sum(e, axis=1, keepdims=True)

    a_ref[...] = jnp.zeros_like(a_ref)
    for h in range(HEADS):
        a_ref[h * CH:(h + 1) * CH, h * CH:(h + 1) * CH] = p[h * CH:(h + 1) * CH, :]


def _k5(x_ref, v_ref, abd_ref, pow_ref, nw_ref, nb_ref, piw_ref,
        x1_ref, h0_ref):
    av = jnp.dot(abd_ref[...].astype(jnp.bfloat16), v_ref[...],
                 preferred_element_type=jnp.float32)
    x1 = x_ref[...] + jnp.dot(pow_ref[...], av, preferred_element_type=jnp.float32)
    x1_ref[...] = x1.astype(jnp.bfloat16)
    xn1 = _ln(x1, nw_ref[...], nb_ref[...])
    h0_ref[...] = jnp.dot(piw_ref[...], xn1.astype(jnp.bfloat16),
                          preferred_element_type=jnp.float32).astype(jnp.bfloat16)


def _k6(a_ref, b_ref, wa_ref, wb_ref, w1_ref, w2_ref, m_ref,
        m1_ref, p1_ref, sa_ref, sb_ref, n1_ref, q1_ref):
    c = a_ref.shape[0]
    wsa = _smem_taps(wa_ref, c)
    _shifts_only(a_ref, m1_ref, p1_ref)
    for ci, r0, acc in _dw_strips(a_ref, m1_ref, p1_ref, wsa):
        sa_ref[ci, r0:r0 + STRIP, :] = acc
    wsb = _smem_taps(wb_ref, c)
    _shifts_only(b_ref, m1_ref, p1_ref)
    for ci, r0, acc in _dw_strips(b_ref, m1_ref, p1_ref, wsb):
        sb_ref[ci, r0:r0 + STRIP, :] = acc
    ws1 = _smem_taps(w1_ref, c)
    ws2 = _smem_taps(w2_ref, c)
    _shifts_only(sa_ref, m1_ref, p1_ref)
    _shifts_only(sb_ref, n1_ref, q1_ref)
    for (ci, r0, t1), (_, _, t2) in zip(
            _dw_strips(sa_ref, m1_ref, p1_ref, ws1),
            _dw_strips(sb_ref, n1_ref, q1_ref, ws2)):
        v1 = jnp.tanh(t1) + sa_ref[ci, r0:r0 + STRIP, :]
        v2 = jnp.tanh(t2) + sb_ref[ci, r0:r0 + STRIP, :]
        m_ref[ci, r0:r0 + STRIP, :] = (v1 * v2).astype(m_ref.dtype)


def _k7(x1_ref, m_ref, ipow_ref, o_ref):
    o_ref[...] = x1_ref[...].astype(jnp.float32) + jnp.dot(
        ipow_ref[...], m_ref[...], preferred_element_type=jnp.float32)


def kernel(x, norm_w, norm_b, temp, q_w, q_dw, kv_w, kv_dw, po_w, g1_w, g1_b,
           g2_w, g2_b, pi_w, dw_w, dw1_w, dw2_w, ipo_w):
    f32 = jnp.float32
    x2d = x.reshape(DIM, N)
    nw = norm_w.reshape(DIM, 1)
    nb = norm_b.reshape(DIM, 1)
    wqkv = jnp.concatenate([q_w.reshape(DIM, DIM), kv_w.reshape(2 * DIM, DIM)],
                           axis=0)
    g1w = g1_w.reshape(DIM // 2, DIM)
    g1b = g1_b.reshape(DIM // 2, 1)
    g2w = g2_w.reshape(1, DIM // 2)
    g2b = g2_b.reshape(1, 1)
    wdw_qkv = jnp.concatenate([q_dw.reshape(DIM, 9), kv_dw.reshape(2 * DIM, 9)],
                              axis=0).reshape(3 * DIM // CT2, CT2, 9)
    tvec = jnp.repeat(temp.reshape(HEADS), CH).reshape(DIM, 1)
    pow_ = po_w.reshape(DIM, DIM)
    piw = pi_w.reshape(2 * HIDDEN, DIM).astype(jnp.bfloat16)
    dwa = dw_w.reshape(2 * HIDDEN, 9)[:HIDDEN].reshape(HIDDEN // CT6, CT6, 9)
    dwb = dw_w.reshape(2 * HIDDEN, 9)[HIDDEN:].reshape(HIDDEN // CT6, CT6, 9)
    dw1 = dw1_w.reshape(HIDDEN // CT6, CT6, 9)
    dw2 = dw2_w.reshape(HIDDEN // CT6, CT6, 9)
    ipow = ipo_w.reshape(DIM, HIDDEN).astype(jnp.bfloat16)

    qkv0, gsum = pl.pallas_call(
        _k1,
        grid=(GN,),
        in_specs=[
            pl.BlockSpec((DIM, NT), lambda i: (0, i)),
            pl.BlockSpec((DIM, 1), lambda i: (0, 0)),
            pl.BlockSpec((DIM, 1), lambda i: (0, 0)),
            pl.BlockSpec((3 * DIM, DIM), lambda i: (0, 0)),
            pl.BlockSpec((DIM // 2, DIM), lambda i: (0, 0)),
            pl.BlockSpec((DIM // 2, 1), lambda i: (0, 0)),
            pl.BlockSpec((1, DIM // 2), lambda i: (0, 0)),
            pl.BlockSpec((1, 1), lambda i: (0, 0)),
        ],
        out_specs=[
            pl.BlockSpec((3 * DIM, NT), lambda i: (0, i)),
            pl.BlockSpec((8, 128), lambda i: (0, 0)),
        ],
        out_shape=[
            jax.ShapeDtypeStruct((3 * DIM, N), jnp.bfloat16),
            jax.ShapeDtypeStruct((8, 128), f32),
        ],
        compiler_params=_ARB,
    )(x2d, nw, nb, wqkv, g1w, g1b, g2w, g2b)

    qkv = pl.pallas_call(
        _k2,
        grid=(3 * DIM // CT2,),
        in_specs=[
            pl.BlockSpec((CT2, HW, HW), lambda c: (c, 0, 0)),
            pl.BlockSpec((1, CT2, 9), lambda c: (c, 0, 0),
                         memory_space=pltpu.SMEM),
        ],
        out_specs=pl.BlockSpec((CT2, HW, HW), lambda c: (c, 0, 0)),
        out_shape=jax.ShapeDtypeStruct((3 * DIM, HW, HW), jnp.bfloat16),
        scratch_shapes=[pltpu.VMEM((CT2, HW, HW), jnp.bfloat16)] * 2,
        compiler_params=_PAR,
    )(qkv0.reshape(3 * DIM, HW, HW), wdw_qkv)
    qkv2d = qkv.reshape(3 * DIM, N)

    gram, stats = pl.pallas_call(
        _k3,
        grid=(GN,),
        in_specs=[
            pl.BlockSpec((DIM, NT), lambda i: (0, i)),
            pl.BlockSpec((DIM, NT), lambda i: (1, i)),
        ],
        out_specs=[
            pl.BlockSpec((DIM, DIM), lambda i: (0, 0)),
            pl.BlockSpec((DIM, 128), lambda i: (0, 0)),
        ],
        out_shape=[
            jax.ShapeDtypeStruct((DIM, DIM), f32),
            jax.ShapeDtypeStruct((DIM, 128), f32),
        ],
        compiler_params=_ARB,
    )(qkv2d, qkv2d)

    abd = pl.pallas_call(
        _k4,
        out_shape=jax.ShapeDtypeStruct((DIM, DIM), f32),
    )(gram, stats, stats.T, tvec, gsum)

    x1, h0 = pl.pallas_call(
        _k5,
        grid=(GN,),
        in_specs=[
            pl.BlockSpec((DIM, NT), lambda i: (0, i)),
            pl.BlockSpec((DIM, NT), lambda i: (2, i)),
            pl.BlockSpec((DIM, DIM), lambda i: (0, 0)),
            pl.BlockSpec((DIM, DIM), lambda i: (0, 0)),
            pl.BlockSpec((DIM, 1), lambda i: (0, 0)),
            pl.BlockSpec((DIM, 1), lambda i: (0, 0)),
            pl.BlockSpec((2 * HIDDEN, DIM), lambda i: (0, 0)),
        ],
        out_specs=[
            pl.BlockSpec((DIM, NT), lambda i: (0, i)),
            pl.BlockSpec((2 * HIDDEN, NT), lambda i: (0, i)),
        ],
        out_shape=[
            jax.ShapeDtypeStruct((DIM, N), jnp.bfloat16),
            jax.ShapeDtypeStruct((2 * HIDDEN, N), jnp.bfloat16),
        ],
        compiler_params=_PAR,
    )(x2d, qkv2d, abd, pow_, nw, nb, piw)

    m = pl.pallas_call(
        _k6,
        grid=(HIDDEN // CT6,),
        in_specs=[
            pl.BlockSpec((CT6, HW, HW), lambda c: (c, 0, 0)),
            pl.BlockSpec((CT6, HW, HW), lambda c: (c + HIDDEN // CT6, 0, 0)),
            pl.BlockSpec((1, CT6, 9), lambda c: (c, 0, 0),
                         memory_space=pltpu.SMEM),
            pl.BlockSpec((1, CT6, 9), lambda c: (c, 0, 0),
                         memory_space=pltpu.SMEM),
            pl.BlockSpec((1, CT6, 9), lambda c: (c, 0, 0),
                         memory_space=pltpu.SMEM),
            pl.BlockSpec((1, CT6, 9), lambda c: (c, 0, 0),
                         memory_space=pltpu.SMEM),
        ],
        out_specs=pl.BlockSpec((CT6, HW, HW), lambda c: (c, 0, 0)),
        out_shape=jax.ShapeDtypeStruct((HIDDEN, HW, HW), jnp.bfloat16),
        scratch_shapes=[pltpu.VMEM((CT6, HW, HW), jnp.bfloat16)] * 6,
        compiler_params=_PAR,
    )(h0.reshape(2 * HIDDEN, HW, HW), h0.reshape(2 * HIDDEN, HW, HW),
      dwa, dwb, dw1, dw2)

    out = pl.pallas_call(
        _k7,
        grid=(GN,),
        in_specs=[
            pl.BlockSpec((DIM, NT), lambda i: (0, i)),
            pl.BlockSpec((HIDDEN, NT), lambda i: (0, i)),
            pl.BlockSpec((DIM, HIDDEN), lambda i: (0, 0)),
        ],
        out_specs=pl.BlockSpec((DIM, NT), lambda i: (0, i)),
        out_shape=jax.ShapeDtypeStruct((DIM, N), f32),
        compiler_params=_PAR,
    )(x1, m.reshape(HIDDEN, N), ipow)

    return out.reshape(1, DIM, HW, HW)
